# baseline (speedup 1.0000x reference)
import functools

import jax
import jax.numpy as jnp
from jax import lax
from jax.experimental import pallas as pl
from jax.experimental.pallas import tpu as pltpu
from jax.experimental.pallas import tpu_sc as plsc

N_U = 10000
N_I = 10000
NP = 10240
E = 320000
D = 128
NB = 10
NBP = 16
B = 1024
TEMP = 0.2
L1 = 0.2
L2 = 0.2
L3 = 1e-7

NCORES = 2
NSUB = 16
NW = NCORES * NSUB

E_PER_SUB = E // NSUB
E_PER_W = E // NW
CH = 80
NCH_SUB = E_PER_SUB // CH
NCH_W = E_PER_W // CH
ROWS_SUB = NP // NSUB
ROWS_CHUNK = 32
NOUT_CH = ROWS_SUB // ROWS_CHUNK
B_PER_W = B // NW

_mesh = plsc.VectorSubcoreMesh(core_axis_name="c", subcore_axis_name="s")


def _zero_vmem(buf, nrows):
    z = jnp.zeros((16,), jnp.float32)

    def body(i, _):
        for d in range(D // 16):
            buf[i, pl.ds(d * 16, 16)] = z
        return 0

    lax.fori_loop(0, nrows, body, 0, unroll=False)


def _scale_rows(rowbuf, vbuf):
    def body(g, _):
        vg = vbuf[pl.ds(g * 16, 16)]
        for e16 in range(16):
            e = g * 16 + e16
            vj = jnp.full((16,), vg[e16], jnp.float32)
            for d in range(D // 16):
                rowbuf[e, pl.ds(d * 16, 16)] = (
                    rowbuf[e, pl.ds(d * 16, 16)] * vj)
        return 0

    lax.fori_loop(0, CH // 16, body, 0, unroll=False)


def _edge_accumulate(tab_hbm, gidx_hbm, sidx_hbm, vals_hbm, acc, slots, obuf,
                     sub):
    base = sub * E_PER_SUB
    _zero_vmem(obuf, ROWS_CHUNK)
    for p in range(NOUT_CH):
        pltpu.sync_copy(obuf, acc.at[pl.ds(sub * ROWS_SUB + p * ROWS_CHUNK,
                                           ROWS_CHUNK)])
    plsc.subcore_barrier()

    def sync_idx(j, sl):
        gidx, sidx, sctx, vbuf, rowbuf, isem, gsem, ssem = sl
        off = base + j * CH
        pltpu.sync_copy(gidx_hbm.at[pl.ds(off, CH)], gidx)
        pltpu.sync_copy(sidx_hbm.at[pl.ds(off, CH)], sidx)
        pltpu.sync_copy(vals_hbm.at[pl.ds(off, CH)], vbuf)

    def fire_idx(j, sl):
        gidx, sidx, sctx, vbuf, rowbuf, isem, gsem, ssem = sl
        off = base + j * CH
        pltpu.async_copy(gidx_hbm.at[pl.ds(off, CH)], gidx, isem)
        pltpu.async_copy(sidx_hbm.at[pl.ds(off, CH)], sidx, isem)
        pltpu.async_copy(vals_hbm.at[pl.ds(off, CH)], vbuf, isem)

    def wait_idx(sl):
        gidx, sidx, sctx, vbuf, rowbuf, isem, gsem, ssem = sl
        pltpu.make_async_copy(gidx_hbm.at[pl.ds(base, CH)], gidx, isem).wait()
        pltpu.make_async_copy(sidx_hbm.at[pl.ds(base, CH)], sidx, isem).wait()
        pltpu.make_async_copy(vals_hbm.at[pl.ds(base, CH)], vbuf, isem).wait()

    def fire_gather(sl):
        gidx, sidx, sctx, vbuf, rowbuf, isem, gsem, ssem = sl
        pltpu.async_copy(tab_hbm.at[gidx], rowbuf, gsem)

    def wait_gather(sl):
        gidx, sidx, sctx, vbuf, rowbuf, isem, gsem, ssem = sl
        pltpu.make_async_copy(tab_hbm.at[gidx], rowbuf, gsem).wait()

    def fire_scatter(sl):
        gidx, sidx, sctx, vbuf, rowbuf, isem, gsem, ssem = sl
        pltpu.async_copy(rowbuf, acc.at[sctx], ssem, add=True)

    def wait_scatter(sl):
        gidx, sidx, sctx, vbuf, rowbuf, isem, gsem, ssem = sl
        pltpu.make_async_copy(rowbuf, acc.at[sctx], ssem).wait()

    sync_idx(0, slots[0])
    sync_idx(1, slots[1])
    fire_gather(slots[0])

    def pair(jj, _):
        for b in range(2):
            j = 2 * jj + b
            sl = slots[b]
            ot = slots[1 - b]
            gidx, sidx, sctx, vbuf, rowbuf, isem, gsem, ssem = sl

            @pl.when((j >= 1) & (j + 1 < NCH_SUB))
            def _():
                wait_scatter(ot)
                wait_idx(ot)

            @pl.when(j + 1 < NCH_SUB)
            def _():
                fire_gather(ot)

            wait_gather(sl)
            for g in range(CH // 16):
                s16 = pl.ds(g * 16, 16)
                sctx[s16] = sidx[s16]
            _scale_rows(rowbuf, vbuf)
            fire_scatter(sl)

            @pl.when(j + 2 < NCH_SUB)
            def _():
                fire_idx(j + 2, sl)
        return 0

    lax.fori_loop(0, NCH_SUB // 2, pair, 0, unroll=False)
    wait_scatter(slots[0])
    wait_scatter(slots[1])
    plsc.subcore_barrier()


def _write_out_plain(acc, obuf, out_hbm, sub):
    for p in range(NOUT_CH):
        sl = pl.ds(sub * ROWS_SUB + p * ROWS_CHUNK, ROWS_CHUNK)
        pltpu.sync_copy(acc.at[sl], obuf)
        pltpu.sync_copy(obuf, out_hbm.at[sl])


def _write_out_fused(acc, obuf, bbuf, pbuf, base_hbm, prev_hbm, out_hbm, sub):
    for p in range(NOUT_CH):
        sl = pl.ds(sub * ROWS_SUB + p * ROWS_CHUNK, ROWS_CHUNK)
        pltpu.sync_copy(acc.at[sl], obuf)
        pltpu.sync_copy(base_hbm.at[sl], bbuf)
        pltpu.sync_copy(prev_hbm.at[sl], pbuf)

        def body(i, _):
            for d in range(D // 16):
                s = pl.ds(d * 16, 16)
                obuf[i, s] = obuf[i, s] + bbuf[i, s] + pbuf[i, s]
            return 0

        lax.fori_loop(0, ROWS_CHUNK, body, 0, unroll=False)
        pltpu.sync_copy(obuf, out_hbm.at[sl])


def _edge_pass_l1_body(rows_hbm, cols_hbm, vals_hbm, tabu_hbm, tabv_hbm,
                       outu_hbm, outv_hbm, acc,
                       g0, s0, c0, v0, r0, i0, gs0, ss0,
                       g1, s1, c1, v1, r1, i1, gs1, ss1, obuf):
    cid = lax.axis_index("c")
    sub = lax.axis_index("s")
    slots = ((g0, s0, c0, v0, r0, i0, gs0, ss0),
             (g1, s1, c1, v1, r1, i1, gs1, ss1))

    @pl.when(cid == 0)
    def _():
        _edge_accumulate(tabu_hbm, cols_hbm, rows_hbm, vals_hbm, acc, slots,
                         obuf, sub)
        _write_out_plain(acc, obuf, outu_hbm, sub)

    @pl.when(cid == 1)
    def _():
        _edge_accumulate(tabv_hbm, rows_hbm, cols_hbm, vals_hbm, acc, slots,
                         obuf, sub)
        _write_out_plain(acc, obuf, outv_hbm, sub)


def _edge_pass_l2_body(rows_hbm, cols_hbm, vals_hbm, tabu_hbm, tabv_hbm,
                       baseu_hbm, basev_hbm, outu_hbm, outv_hbm, acc,
                       g0, s0, c0, v0, r0, i0, gs0, ss0,
                       g1, s1, c1, v1, r1, i1, gs1, ss1, obuf, bbuf, pbuf):
    cid = lax.axis_index("c")
    sub = lax.axis_index("s")
    slots = ((g0, s0, c0, v0, r0, i0, gs0, ss0),
             (g1, s1, c1, v1, r1, i1, gs1, ss1))

    @pl.when(cid == 0)
    def _():
        _edge_accumulate(tabu_hbm, cols_hbm, rows_hbm, vals_hbm, acc, slots,
                         obuf, sub)
        _write_out_fused(acc, obuf, bbuf, pbuf, baseu_hbm, tabv_hbm, outu_hbm,
                         sub)

    @pl.when(cid == 1)
    def _():
        _edge_accumulate(tabv_hbm, rows_hbm, cols_hbm, vals_hbm, acc, slots,
                         obuf, sub)
        _write_out_fused(acc, obuf, bbuf, pbuf, basev_hbm, tabu_hbm, outv_hbm,
                         sub)


def _slot_scratch():
    return [
        pltpu.VMEM((CH,), jnp.int32),
        pltpu.VMEM((CH,), jnp.int32),
        pltpu.VMEM((CH,), jnp.int32),
        pltpu.VMEM((CH,), jnp.float32),
        pltpu.VMEM((CH, D), jnp.float32),
        pltpu.SemaphoreType.DMA,
        pltpu.SemaphoreType.DMA,
        pltpu.SemaphoreType.DMA,
    ]


_SCRATCH_COMMON = (
    [pltpu.VMEM_SHARED((NP, D), jnp.float32)]
    + _slot_scratch() + _slot_scratch()
    + [pltpu.VMEM((ROWS_CHUNK, D), jnp.float32)]
)

_edge_pass_l1 = pl.kernel(
    _edge_pass_l1_body,
    out_type=(jax.ShapeDtypeStruct((NP, D), jnp.float32),
              jax.ShapeDtypeStruct((NP, D), jnp.float32)),
    mesh=_mesh,
    scratch_types=_SCRATCH_COMMON,
    compiler_params=pltpu.CompilerParams(needs_layout_passes=False),
)

_edge_pass_l2 = pl.kernel(
    _edge_pass_l2_body,
    out_type=(jax.ShapeDtypeStruct((NP, D), jnp.float32),
              jax.ShapeDtypeStruct((NP, D), jnp.float32)),
    mesh=_mesh,
    scratch_types=_SCRATCH_COMMON + [
        pltpu.VMEM((ROWS_CHUNK, D), jnp.float32),
        pltpu.VMEM((ROWS_CHUNK, D), jnp.float32),
    ],
    compiler_params=pltpu.CompilerParams(needs_layout_passes=False),
)


def _aug_vals_body(rows_hbm, cols_hbm, adj_hbm, eu_hbm, ev_hbm, out_hbm,
                   r0, c0, a0, xu0, xi0, ob0, i0, gs0, os0,
                   r1, c1, a1, xu1, xi1, ob1, i1, gs1, os1):
    wid = lax.axis_index("s") * NCORES + lax.axis_index("c")
    base = wid * E_PER_W
    slots = ((r0, c0, a0, xu0, xi0, ob0, i0, gs0, os0),
             (r1, c1, a1, xu1, xi1, ob1, i1, gs1, os1))

    def sync_idx(j, sl):
        ridx, cidx, abuf, xu, xi, obuf, isem, gsem, osem = sl
        off = base + j * CH
        pltpu.sync_copy(rows_hbm.at[pl.ds(off, CH)], ridx)
        pltpu.sync_copy(cols_hbm.at[pl.ds(off, CH)], cidx)
        pltpu.sync_copy(adj_hbm.at[pl.ds(off, CH)], abuf)

    def fire_idx(j, sl):
        ridx, cidx, abuf, xu, xi, obuf, isem, gsem, osem = sl
        off = base + j * CH
        pltpu.async_copy(rows_hbm.at[pl.ds(off, CH)], ridx, isem)
        pltpu.async_copy(cols_hbm.at[pl.ds(off, CH)], cidx, isem)
        pltpu.async_copy(adj_hbm.at[pl.ds(off, CH)], abuf, isem)

    def wait_idx(sl):
        ridx, cidx, abuf, xu, xi, obuf, isem, gsem, osem = sl
        pltpu.make_async_copy(rows_hbm.at[pl.ds(base, CH)], ridx, isem).wait()
        pltpu.make_async_copy(cols_hbm.at[pl.ds(base, CH)], cidx, isem).wait()
        pltpu.make_async_copy(adj_hbm.at[pl.ds(base, CH)], abuf, isem).wait()

    def fire_gather(sl):
        ridx, cidx, abuf, xu, xi, obuf, isem, gsem, osem = sl
        pltpu.async_copy(eu_hbm.at[ridx], xu, gsem)
        pltpu.async_copy(ev_hbm.at[cidx], xi, gsem)

    def wait_gather(sl):
        ridx, cidx, abuf, xu, xi, obuf, isem, gsem, osem = sl
        pltpu.make_async_copy(eu_hbm.at[ridx], xu, gsem).wait()
        pltpu.make_async_copy(ev_hbm.at[cidx], xi, gsem).wait()

    def sync_out(j, sl):
        ridx, cidx, abuf, xu, xi, obuf, isem, gsem, osem = sl
        pltpu.sync_copy(obuf, out_hbm.at[pl.ds(base + j * CH, CH)])

    sync_idx(0, slots[0])
    sync_idx(1, slots[1])
    fire_gather(slots[0])
    lane = lax.iota(jnp.int32, 16)

    def pair(jj, _):
        for b in range(2):
            j = 2 * jj + b
            sl = slots[b]
            ot = slots[1 - b]
            ridx, cidx, abuf, xu, xi, obuf, isem, gsem, osem = sl

            @pl.when((j >= 1) & (j + 1 < NCH_W))
            def _():
                wait_idx(ot)

            @pl.when(j + 1 < NCH_W)
            def _():
                fire_gather(ot)

            wait_gather(sl)

            def group(g, _):
                ex = lane + g * 16

                def dblk(dk, accs):
                    a0, a1 = accs
                    for dd in range(0, 16, 2):
                        d0 = jnp.full((16,), dk * 16 + dd, jnp.int32)
                        d1 = jnp.full((16,), dk * 16 + dd + 1, jnp.int32)
                        a0 = a0 + (plsc.load_gather(xu, [ex, d0])
                                   * plsc.load_gather(xi, [ex, d0]))
                        a1 = a1 + (plsc.load_gather(xu, [ex, d1])
                                   * plsc.load_gather(xi, [ex, d1]))
                    return (a0, a1)

                z = jnp.zeros((16,), jnp.float32)
                a0, a1 = lax.fori_loop(0, D // 16, dblk, (z, z))
                dvec = a0 + a1
                s = pl.ds(g * 16, 16)
                obuf[s] = abuf[s] / (1.0 + jnp.exp(-dvec))
                return 0

            lax.fori_loop(0, CH // 16, group, 0, unroll=False)
            sync_out(j, sl)

            @pl.when(j + 2 < NCH_W)
            def _():
                fire_idx(j + 2, sl)
        return 0

    lax.fori_loop(0, NCH_W // 2, pair, 0, unroll=False)


def _aug_slot_scratch():
    return [
        pltpu.VMEM((CH,), jnp.int32),
        pltpu.VMEM((CH,), jnp.int32),
        pltpu.VMEM((CH,), jnp.float32),
        pltpu.VMEM((CH, D), jnp.float32),
        pltpu.VMEM((CH, D), jnp.float32),
        pltpu.VMEM((CH,), jnp.float32),
        pltpu.SemaphoreType.DMA,
        pltpu.SemaphoreType.DMA,
        pltpu.SemaphoreType.DMA,
    ]


_aug_vals = pl.kernel(
    _aug_vals_body,
    out_type=jax.ShapeDtypeStruct((E,), jnp.float32),
    mesh=_mesh,
    scratch_types=_aug_slot_scratch() + _aug_slot_scratch(),
    compiler_params=pltpu.CompilerParams(needs_layout_passes=False),
)


def _gather6_body(eu_hbm, ev_hbm, zu_hbm, zv_hbm, uids_hbm, iids_hbm, pos_hbm,
                  neg_hbm, o_uemb, o_pos, o_neg, o_zub, o_zvb, o_evb,
                  ibuf, rbuf):
    wid = lax.axis_index("s") * NCORES + lax.axis_index("c")
    sl = pl.ds(wid * B_PER_W, B_PER_W)
    for idx_hbm, tab_hbm, out_hbm in (
        (uids_hbm, eu_hbm, o_uemb),
        (pos_hbm, ev_hbm, o_pos),
        (neg_hbm, ev_hbm, o_neg),
        (uids_hbm, zu_hbm, o_zub),
        (iids_hbm, zv_hbm, o_zvb),
        (iids_hbm, ev_hbm, o_evb),
    ):
        pltpu.sync_copy(idx_hbm.at[sl], ibuf)
        pltpu.sync_copy(tab_hbm.at[ibuf], rbuf)
        pltpu.sync_copy(rbuf, out_hbm.at[sl])


_gather6 = pl.kernel(
    _gather6_body,
    out_type=tuple(jax.ShapeDtypeStruct((B, D), jnp.float32)
                   for _ in range(6)),
    mesh=_mesh,
    scratch_types=[
        pltpu.VMEM((B_PER_W,), jnp.int32),
        pltpu.VMEM((B_PER_W, D), jnp.float32),
    ],
    compiler_params=pltpu.CompilerParams(needs_layout_passes=False),
)


def _losses_body(eu_ref, ev_ref, eu0_ref, ev0_ref, ebp_ref, uemb_ref, pos_ref,
                 neg_ref, zub_ref, zvb_ref, evb_ref, out_ref):
    u_emb = uemb_ref[...]
    pos_emb = pos_ref[...]
    neg_emb = neg_ref[...]
    zub = zub_ref[...]
    zvb = zvb_ref[...]
    evb = evb_ref[...]

    pos_scores = jnp.sum(u_emb * pos_emb, axis=1, keepdims=True)
    neg_scores = jnp.sum(u_emb * neg_emb, axis=1, keepdims=True)
    diff = pos_scores - neg_scores
    sig = 1.0 / (1.0 + jnp.exp(-diff))
    loss_bpr = -jnp.sum(jnp.log(sig)) / B

    def pcl_neg(zb, tab_ref):
        def blk(k, acc):
            t = tab_ref[pl.ds(k * 1000, 1000), :]
            s = lax.dot_general(zb, t, (((1,), (1,)), ((), ())),
                                preferred_element_type=jnp.float32)
            return acc + jnp.sum(jnp.exp(s / TEMP), axis=1, keepdims=True)

        acc = lax.fori_loop(0, N_U // 1000, blk,
                            jnp.zeros((B, 1), jnp.float32))
        return jnp.sum(jnp.log(acc + 1e-8)) / B

    neg_s = pcl_neg(zub, eu_ref) + pcl_neg(zvb, ev_ref)
    pos_s = (jnp.sum(jnp.clip(jnp.sum(zub * u_emb, axis=1) / TEMP, -5.0, 5.0))
             / B
             + jnp.sum(jnp.clip(jnp.sum(zvb * evb, axis=1) / TEMP, -5.0, 5.0))
             / B)
    loss_pcl = -pos_s + neg_s

    ps_min = jnp.min(pos_scores)
    ps_max = jnp.max(pos_scores)
    weight_b = (pos_scores - ps_min) / (ps_max - ps_min + 1e-9)
    relations = jnp.clip((weight_b * NB).astype(jnp.int32), 0, NB - 1)
    el = 1.0 / (1.0 + jnp.exp(-(u_emb * pos_emb)))
    s_all = lax.dot_general(el, ebp_ref[...], (((1,), (1,)), ((), ())),
                            preferred_element_type=jnp.float32)
    lane = lax.broadcasted_iota(jnp.int32, (B, NBP), 1)
    onehot = lane == relations
    srel = jnp.sum(jnp.where(onehot, s_all, 0.0), axis=1, keepdims=True)
    ssum = jnp.sum(s_all, axis=1, keepdims=True)
    neg_bcl = jnp.sum((ssum - srel) / NB) / B
    pos_bcl = jnp.sum(srel) / B
    loss_bcl = neg_bcl - pos_bcl

    def sq(tab_ref):
        def blk(k, acc):
            t = tab_ref[pl.ds(k * 200, 200), :]
            return acc + jnp.sum(t * t)

        return lax.fori_loop(0, N_U // 200, blk, jnp.float32(0.0))

    loss_reg = L3 * (sq(eu0_ref) + sq(ev0_ref) + jnp.sum(ebp_ref[...] ** 2))

    loss = loss_bpr + L1 * loss_pcl + L2 * loss_bcl + loss_reg
    out_ref[0] = loss
    out_ref[1] = loss_bpr
    out_ref[2] = L1 * loss_pcl
    out_ref[3] = L2 * loss_bcl


def _losses_call(eu, ev, eu0, ev0, ebp, uemb, posb, negb, zub, zvb, evb):
    return pl.pallas_call(
        _losses_body,
        out_shape=jax.ShapeDtypeStruct((4,), jnp.float32),
        in_specs=[pl.BlockSpec(memory_space=pltpu.VMEM)] * 11,
        out_specs=pl.BlockSpec(memory_space=pltpu.SMEM),
    )(eu, ev, eu0, ev0, ebp, uemb, posb, negb, zub, zvb, evb)


def kernel(E_u_0, E_v_0, E_b, adj_vals, edgE_vndex, uids, iids, pos, neg):
    rows = edgE_vndex[0]
    cols = edgE_vndex[1]
    pad = ((0, NP - N_U), (0, 0))
    eu0p = jnp.pad(E_u_0, pad)
    ev0p = jnp.pad(E_v_0, pad)

    nu1, nv1 = _edge_pass_l1(rows, cols, adj_vals, ev0p, eu0p)
    E_u, E_v = _edge_pass_l2(rows, cols, adj_vals, nv1, nu1, eu0p, ev0p)

    aug = _aug_vals(rows, cols, adj_vals, E_u, E_v)

    m_u1, m_v1 = _edge_pass_l1(rows, cols, aug, ev0p, eu0p)
    Z_u, Z_v = _edge_pass_l2(rows, cols, aug, m_v1, m_u1, eu0p, ev0p)

    u_emb, pos_emb, neg_emb, zub, zvb, evb = _gather6(
        E_u, E_v, Z_u, Z_v, uids, iids, pos, neg)

    ebp = jnp.zeros((NBP, D), jnp.float32).at[:NB].set(E_b)
    out = _losses_call(E_u, E_v, E_u_0, E_v_0, ebp, u_emb, pos_emb, neg_emb,
                       zub, zvb, evb)
    return (out[0], out[1], out[2], out[3])


# speedup vs baseline: 1.3054x; 1.3054x over previous
import functools

import jax
import jax.numpy as jnp
from jax import lax
from jax.experimental import pallas as pl
from jax.experimental.pallas import tpu as pltpu
from jax.experimental.pallas import tpu_sc as plsc

N_U = 10000
N_I = 10000
NP = 10240
E = 320000
D = 128
NB = 10
NBP = 16
B = 1024
TEMP = 0.2
L1 = 0.2
L2 = 0.2
L3 = 1e-7

NCORES = 2
NSUB = 16
NW = NCORES * NSUB

E_PER_SUB = E // NSUB
E_PER_W = E // NW
CH = 80
NCH_SUB = E_PER_SUB // CH
NCH_W = E_PER_W // CH
ROWS_SUB = NP // NSUB
ROWS_CHUNK = 32
NOUT_CH = ROWS_SUB // ROWS_CHUNK
B_PER_W = B // NW

_mesh = plsc.VectorSubcoreMesh(core_axis_name="c", subcore_axis_name="s")


def _zero_vmem(buf, nrows):
    z = jnp.zeros((16,), jnp.float32)

    def body(i, _):
        for d in range(D // 16):
            buf[i, pl.ds(d * 16, 16)] = z
        return 0

    lax.fori_loop(0, nrows, body, 0, unroll=False)


def _scale_rows(rowbuf, vbuf):
    def body(g, _):
        vg = vbuf[pl.ds(g * 16, 16)]
        for e16 in range(16):
            e = g * 16 + e16
            vj = jnp.full((16,), vg[e16], jnp.float32)
            for d in range(D // 16):
                rowbuf[e, pl.ds(d * 16, 16)] = (
                    rowbuf[e, pl.ds(d * 16, 16)] * vj)
        return 0

    lax.fori_loop(0, CH // 16, body, 0, unroll=False)


def _edge_accumulate(tab_hbm, gidx_hbm, sidx_hbm, vals_hbm, acc, slots, obuf,
                     sub):
    base = sub * E_PER_SUB
    _zero_vmem(obuf, ROWS_CHUNK)
    for p in range(NOUT_CH):
        pltpu.sync_copy(obuf, acc.at[pl.ds(sub * ROWS_SUB + p * ROWS_CHUNK,
                                           ROWS_CHUNK)])
    plsc.subcore_barrier()

    def sync_idx(j, sl):
        gidx, sidx, sctx, vbuf, rowbuf, isem, gsem, ssem = sl
        off = base + j * CH
        pltpu.sync_copy(gidx_hbm.at[pl.ds(off, CH)], gidx)
        pltpu.sync_copy(sidx_hbm.at[pl.ds(off, CH)], sidx)
        pltpu.sync_copy(vals_hbm.at[pl.ds(off, CH)], vbuf)

    def fire_idx(j, sl):
        gidx, sidx, sctx, vbuf, rowbuf, isem, gsem, ssem = sl
        off = base + j * CH
        pltpu.async_copy(gidx_hbm.at[pl.ds(off, CH)], gidx, isem)
        pltpu.async_copy(sidx_hbm.at[pl.ds(off, CH)], sidx, isem)
        pltpu.async_copy(vals_hbm.at[pl.ds(off, CH)], vbuf, isem)

    def wait_idx(sl):
        gidx, sidx, sctx, vbuf, rowbuf, isem, gsem, ssem = sl
        pltpu.make_async_copy(gidx_hbm.at[pl.ds(base, CH)], gidx, isem).wait()
        pltpu.make_async_copy(sidx_hbm.at[pl.ds(base, CH)], sidx, isem).wait()
        pltpu.make_async_copy(vals_hbm.at[pl.ds(base, CH)], vbuf, isem).wait()

    def fire_gather(sl):
        gidx, sidx, sctx, vbuf, rowbuf, isem, gsem, ssem = sl
        pltpu.async_copy(tab_hbm.at[gidx], rowbuf, gsem)

    def wait_gather(sl):
        gidx, sidx, sctx, vbuf, rowbuf, isem, gsem, ssem = sl
        pltpu.make_async_copy(tab_hbm.at[gidx], rowbuf, gsem).wait()

    def fire_scatter(sl):
        gidx, sidx, sctx, vbuf, rowbuf, isem, gsem, ssem = sl
        pltpu.async_copy(rowbuf, acc.at[sctx], ssem, add=True)

    def wait_scatter(sl):
        gidx, sidx, sctx, vbuf, rowbuf, isem, gsem, ssem = sl
        pltpu.make_async_copy(rowbuf, acc.at[sctx], ssem).wait()

    sync_idx(0, slots[0])
    sync_idx(1, slots[1])
    fire_gather(slots[0])

    def pair(jj, _):
        for b in range(2):
            j = 2 * jj + b
            sl = slots[b]
            ot = slots[1 - b]
            gidx, sidx, sctx, vbuf, rowbuf, isem, gsem, ssem = sl

            @pl.when((j >= 1) & (j + 1 < NCH_SUB))
            def _():
                wait_scatter(ot)
                wait_idx(ot)

            @pl.when(j + 1 < NCH_SUB)
            def _():
                fire_gather(ot)

            wait_gather(sl)
            for g in range(CH // 16):
                s16 = pl.ds(g * 16, 16)
                sctx[s16] = sidx[s16]
            _scale_rows(rowbuf, vbuf)
            fire_scatter(sl)

            @pl.when(j + 2 < NCH_SUB)
            def _():
                fire_idx(j + 2, sl)
        return 0

    lax.fori_loop(0, NCH_SUB // 2, pair, 0, unroll=False)
    wait_scatter(slots[0])
    wait_scatter(slots[1])
    plsc.subcore_barrier()


def _write_out_plain(acc, obuf, out_hbm, sub):
    for p in range(NOUT_CH):
        sl = pl.ds(sub * ROWS_SUB + p * ROWS_CHUNK, ROWS_CHUNK)
        pltpu.sync_copy(acc.at[sl], obuf)
        pltpu.sync_copy(obuf, out_hbm.at[sl])


def _write_out_fused(acc, obuf, bbuf, pbuf, base_hbm, prev_hbm, out_hbm, sub):
    for p in range(NOUT_CH):
        sl = pl.ds(sub * ROWS_SUB + p * ROWS_CHUNK, ROWS_CHUNK)
        pltpu.sync_copy(acc.at[sl], obuf)
        pltpu.sync_copy(base_hbm.at[sl], bbuf)
        pltpu.sync_copy(prev_hbm.at[sl], pbuf)

        def body(i, _):
            for d in range(D // 16):
                s = pl.ds(d * 16, 16)
                obuf[i, s] = obuf[i, s] + bbuf[i, s] + pbuf[i, s]
            return 0

        lax.fori_loop(0, ROWS_CHUNK, body, 0, unroll=False)
        pltpu.sync_copy(obuf, out_hbm.at[sl])


def _edge_pass_l1_body(rows_hbm, cols_hbm, vals_hbm, tabu_hbm, tabv_hbm,
                       outu_hbm, outv_hbm, acc,
                       g0, s0, c0, v0, r0, i0, gs0, ss0,
                       g1, s1, c1, v1, r1, i1, gs1, ss1, obuf):
    cid = lax.axis_index("c")
    sub = lax.axis_index("s")
    slots = ((g0, s0, c0, v0, r0, i0, gs0, ss0),
             (g1, s1, c1, v1, r1, i1, gs1, ss1))

    @pl.when(cid == 0)
    def _():
        _edge_accumulate(tabu_hbm, cols_hbm, rows_hbm, vals_hbm, acc, slots,
                         obuf, sub)
        _write_out_plain(acc, obuf, outu_hbm, sub)

    @pl.when(cid == 1)
    def _():
        _edge_accumulate(tabv_hbm, rows_hbm, cols_hbm, vals_hbm, acc, slots,
                         obuf, sub)
        _write_out_plain(acc, obuf, outv_hbm, sub)


def _edge_pass_l2_body(rows_hbm, cols_hbm, vals_hbm, tabu_hbm, tabv_hbm,
                       baseu_hbm, basev_hbm, outu_hbm, outv_hbm, acc,
                       g0, s0, c0, v0, r0, i0, gs0, ss0,
                       g1, s1, c1, v1, r1, i1, gs1, ss1, obuf, bbuf, pbuf):
    cid = lax.axis_index("c")
    sub = lax.axis_index("s")
    slots = ((g0, s0, c0, v0, r0, i0, gs0, ss0),
             (g1, s1, c1, v1, r1, i1, gs1, ss1))

    @pl.when(cid == 0)
    def _():
        _edge_accumulate(tabu_hbm, cols_hbm, rows_hbm, vals_hbm, acc, slots,
                         obuf, sub)
        _write_out_fused(acc, obuf, bbuf, pbuf, baseu_hbm, tabv_hbm, outu_hbm,
                         sub)

    @pl.when(cid == 1)
    def _():
        _edge_accumulate(tabv_hbm, rows_hbm, cols_hbm, vals_hbm, acc, slots,
                         obuf, sub)
        _write_out_fused(acc, obuf, bbuf, pbuf, basev_hbm, tabu_hbm, outv_hbm,
                         sub)


def _slot_scratch():
    return [
        pltpu.VMEM((CH,), jnp.int32),
        pltpu.VMEM((CH,), jnp.int32),
        pltpu.VMEM((CH,), jnp.int32),
        pltpu.VMEM((CH,), jnp.float32),
        pltpu.VMEM((CH, D), jnp.float32),
        pltpu.SemaphoreType.DMA,
        pltpu.SemaphoreType.DMA,
        pltpu.SemaphoreType.DMA,
    ]


_SCRATCH_COMMON = (
    [pltpu.VMEM_SHARED((NP, D), jnp.float32)]
    + _slot_scratch() + _slot_scratch()
    + [pltpu.VMEM((ROWS_CHUNK, D), jnp.float32)]
)

_edge_pass_l1 = pl.kernel(
    _edge_pass_l1_body,
    out_type=(jax.ShapeDtypeStruct((NP, D), jnp.float32),
              jax.ShapeDtypeStruct((NP, D), jnp.float32)),
    mesh=_mesh,
    scratch_types=_SCRATCH_COMMON,
    compiler_params=pltpu.CompilerParams(needs_layout_passes=False),
)

_edge_pass_l2 = pl.kernel(
    _edge_pass_l2_body,
    out_type=(jax.ShapeDtypeStruct((NP, D), jnp.float32),
              jax.ShapeDtypeStruct((NP, D), jnp.float32)),
    mesh=_mesh,
    scratch_types=_SCRATCH_COMMON + [
        pltpu.VMEM((ROWS_CHUNK, D), jnp.float32),
        pltpu.VMEM((ROWS_CHUNK, D), jnp.float32),
    ],
    compiler_params=pltpu.CompilerParams(needs_layout_passes=False),
)


def _aug_vals_body(rows_hbm, cols_hbm, adj_hbm, eu_hbm, ev_hbm, out_hbm,
                   r0, c0, a0, xu0, xi0, ob0, i0, gs0, os0,
                   r1, c1, a1, xu1, xi1, ob1, i1, gs1, os1):
    wid = lax.axis_index("s") * NCORES + lax.axis_index("c")
    base = wid * E_PER_W
    slots = ((r0, c0, a0, xu0, xi0, ob0, i0, gs0, os0),
             (r1, c1, a1, xu1, xi1, ob1, i1, gs1, os1))

    def sync_idx(j, sl):
        ridx, cidx, abuf, xu, xi, obuf, isem, gsem, osem = sl
        off = base + j * CH
        pltpu.sync_copy(rows_hbm.at[pl.ds(off, CH)], ridx)
        pltpu.sync_copy(cols_hbm.at[pl.ds(off, CH)], cidx)
        pltpu.sync_copy(adj_hbm.at[pl.ds(off, CH)], abuf)

    def fire_idx(j, sl):
        ridx, cidx, abuf, xu, xi, obuf, isem, gsem, osem = sl
        off = base + j * CH
        pltpu.async_copy(rows_hbm.at[pl.ds(off, CH)], ridx, isem)
        pltpu.async_copy(cols_hbm.at[pl.ds(off, CH)], cidx, isem)
        pltpu.async_copy(adj_hbm.at[pl.ds(off, CH)], abuf, isem)

    def wait_idx(sl):
        ridx, cidx, abuf, xu, xi, obuf, isem, gsem, osem = sl
        pltpu.make_async_copy(rows_hbm.at[pl.ds(base, CH)], ridx, isem).wait()
        pltpu.make_async_copy(cols_hbm.at[pl.ds(base, CH)], cidx, isem).wait()
        pltpu.make_async_copy(adj_hbm.at[pl.ds(base, CH)], abuf, isem).wait()

    def fire_gather(sl):
        ridx, cidx, abuf, xu, xi, obuf, isem, gsem, osem = sl
        pltpu.async_copy(eu_hbm.at[ridx], xu, gsem)
        pltpu.async_copy(ev_hbm.at[cidx], xi, gsem)

    def wait_gather(sl):
        ridx, cidx, abuf, xu, xi, obuf, isem, gsem, osem = sl
        pltpu.make_async_copy(eu_hbm.at[ridx], xu, gsem).wait()
        pltpu.make_async_copy(ev_hbm.at[cidx], xi, gsem).wait()

    def sync_out(j, sl):
        ridx, cidx, abuf, xu, xi, obuf, isem, gsem, osem = sl
        pltpu.sync_copy(obuf, out_hbm.at[pl.ds(base + j * CH, CH)])

    sync_idx(0, slots[0])
    sync_idx(1, slots[1])
    fire_gather(slots[0])
    lane = lax.iota(jnp.int32, 16)

    def pair(jj, _):
        for b in range(2):
            j = 2 * jj + b
            sl = slots[b]
            ot = slots[1 - b]
            ridx, cidx, abuf, xu, xi, obuf, isem, gsem, osem = sl

            @pl.when((j >= 1) & (j + 1 < NCH_W))
            def _():
                wait_idx(ot)

            @pl.when(j + 1 < NCH_W)
            def _():
                fire_gather(ot)

            wait_gather(sl)
            for g in range(CH // 16):
                dvec = jnp.zeros((16,), jnp.float32)
                for e16 in range(16):
                    e = g * 16 + e16
                    acc = xu[e, pl.ds(0, 16)] * xi[e, pl.ds(0, 16)]
                    for d in range(1, D // 16):
                        s = pl.ds(d * 16, 16)
                        acc = acc + xu[e, s] * xi[e, s]
                    for k in (8, 4, 2, 1):
                        acc = acc + jnp.take(acc, lane ^ k)
                    dvec = jnp.where(lane == e16, acc, dvec)
                s = pl.ds(g * 16, 16)
                obuf[s] = abuf[s] / (1.0 + jnp.exp(-dvec))
            sync_out(j, sl)

            @pl.when(j + 2 < NCH_W)
            def _():
                fire_idx(j + 2, sl)
        return 0

    lax.fori_loop(0, NCH_W // 2, pair, 0, unroll=False)


def _aug_slot_scratch():
    return [
        pltpu.VMEM((CH,), jnp.int32),
        pltpu.VMEM((CH,), jnp.int32),
        pltpu.VMEM((CH,), jnp.float32),
        pltpu.VMEM((CH, D), jnp.float32),
        pltpu.VMEM((CH, D), jnp.float32),
        pltpu.VMEM((CH,), jnp.float32),
        pltpu.SemaphoreType.DMA,
        pltpu.SemaphoreType.DMA,
        pltpu.SemaphoreType.DMA,
    ]


_aug_vals = pl.kernel(
    _aug_vals_body,
    out_type=jax.ShapeDtypeStruct((E,), jnp.float32),
    mesh=_mesh,
    scratch_types=_aug_slot_scratch() + _aug_slot_scratch(),
    compiler_params=pltpu.CompilerParams(needs_layout_passes=False),
)


def _gather6_body(eu_hbm, ev_hbm, zu_hbm, zv_hbm, uids_hbm, iids_hbm, pos_hbm,
                  neg_hbm, o_uemb, o_pos, o_neg, o_zub, o_zvb, o_evb,
                  ibuf, rbuf):
    wid = lax.axis_index("s") * NCORES + lax.axis_index("c")
    sl = pl.ds(wid * B_PER_W, B_PER_W)
    for idx_hbm, tab_hbm, out_hbm in (
        (uids_hbm, eu_hbm, o_uemb),
        (pos_hbm, ev_hbm, o_pos),
        (neg_hbm, ev_hbm, o_neg),
        (uids_hbm, zu_hbm, o_zub),
        (iids_hbm, zv_hbm, o_zvb),
        (iids_hbm, ev_hbm, o_evb),
    ):
        pltpu.sync_copy(idx_hbm.at[sl], ibuf)
        pltpu.sync_copy(tab_hbm.at[ibuf], rbuf)
        pltpu.sync_copy(rbuf, out_hbm.at[sl])


_gather6 = pl.kernel(
    _gather6_body,
    out_type=tuple(jax.ShapeDtypeStruct((B, D), jnp.float32)
                   for _ in range(6)),
    mesh=_mesh,
    scratch_types=[
        pltpu.VMEM((B_PER_W,), jnp.int32),
        pltpu.VMEM((B_PER_W, D), jnp.float32),
    ],
    compiler_params=pltpu.CompilerParams(needs_layout_passes=False),
)


def _losses_body(eu_ref, ev_ref, eu0_ref, ev0_ref, ebp_ref, uemb_ref, pos_ref,
                 neg_ref, zub_ref, zvb_ref, evb_ref, out_ref):
    u_emb = uemb_ref[...]
    pos_emb = pos_ref[...]
    neg_emb = neg_ref[...]
    zub = zub_ref[...]
    zvb = zvb_ref[...]
    evb = evb_ref[...]

    pos_scores = jnp.sum(u_emb * pos_emb, axis=1, keepdims=True)
    neg_scores = jnp.sum(u_emb * neg_emb, axis=1, keepdims=True)
    diff = pos_scores - neg_scores
    sig = 1.0 / (1.0 + jnp.exp(-diff))
    loss_bpr = -jnp.sum(jnp.log(sig)) / B

    def pcl_neg(zb, tab_ref):
        def blk(k, acc):
            t = tab_ref[pl.ds(k * 1000, 1000), :]
            s = lax.dot_general(zb, t, (((1,), (1,)), ((), ())),
                                preferred_element_type=jnp.float32)
            return acc + jnp.sum(jnp.exp(s / TEMP), axis=1, keepdims=True)

        acc = lax.fori_loop(0, N_U // 1000, blk,
                            jnp.zeros((B, 1), jnp.float32))
        return jnp.sum(jnp.log(acc + 1e-8)) / B

    neg_s = pcl_neg(zub, eu_ref) + pcl_neg(zvb, ev_ref)
    pos_s = (jnp.sum(jnp.clip(jnp.sum(zub * u_emb, axis=1) / TEMP, -5.0, 5.0))
             / B
             + jnp.sum(jnp.clip(jnp.sum(zvb * evb, axis=1) / TEMP, -5.0, 5.0))
             / B)
    loss_pcl = -pos_s + neg_s

    ps_min = jnp.min(pos_scores)
    ps_max = jnp.max(pos_scores)
    weight_b = (pos_scores - ps_min) / (ps_max - ps_min + 1e-9)
    relations = jnp.clip((weight_b * NB).astype(jnp.int32), 0, NB - 1)
    el = 1.0 / (1.0 + jnp.exp(-(u_emb * pos_emb)))
    s_all = lax.dot_general(el, ebp_ref[...], (((1,), (1,)), ((), ())),
                            preferred_element_type=jnp.float32)
    lane = lax.broadcasted_iota(jnp.int32, (B, NBP), 1)
    onehot = lane == relations
    srel = jnp.sum(jnp.where(onehot, s_all, 0.0), axis=1, keepdims=True)
    ssum = jnp.sum(s_all, axis=1, keepdims=True)
    neg_bcl = jnp.sum((ssum - srel) / NB) / B
    pos_bcl = jnp.sum(srel) / B
    loss_bcl = neg_bcl - pos_bcl

    def sq(tab_ref):
        def blk(k, acc):
            t = tab_ref[pl.ds(k * 200, 200), :]
            return acc + jnp.sum(t * t)

        return lax.fori_loop(0, N_U // 200, blk, jnp.float32(0.0))

    loss_reg = L3 * (sq(eu0_ref) + sq(ev0_ref) + jnp.sum(ebp_ref[...] ** 2))

    loss = loss_bpr + L1 * loss_pcl + L2 * loss_bcl + loss_reg
    out_ref[0] = loss
    out_ref[1] = loss_bpr
    out_ref[2] = L1 * loss_pcl
    out_ref[3] = L2 * loss_bcl


def _losses_call(eu, ev, eu0, ev0, ebp, uemb, posb, negb, zub, zvb, evb):
    return pl.pallas_call(
        _losses_body,
        out_shape=jax.ShapeDtypeStruct((4,), jnp.float32),
        in_specs=[pl.BlockSpec(memory_space=pltpu.VMEM)] * 11,
        out_specs=pl.BlockSpec(memory_space=pltpu.SMEM),
    )(eu, ev, eu0, ev0, ebp, uemb, posb, negb, zub, zvb, evb)


def kernel(E_u_0, E_v_0, E_b, adj_vals, edgE_vndex, uids, iids, pos, neg):
    rows = edgE_vndex[0]
    cols = edgE_vndex[1]
    pad = ((0, NP - N_U), (0, 0))
    eu0p = jnp.pad(E_u_0, pad)
    ev0p = jnp.pad(E_v_0, pad)

    nu1, nv1 = _edge_pass_l1(rows, cols, adj_vals, ev0p, eu0p)
    E_u, E_v = _edge_pass_l2(rows, cols, adj_vals, nv1, nu1, eu0p, ev0p)

    aug = _aug_vals(rows, cols, adj_vals, E_u, E_v)

    m_u1, m_v1 = _edge_pass_l1(rows, cols, aug, ev0p, eu0p)
    Z_u, Z_v = _edge_pass_l2(rows, cols, aug, m_v1, m_u1, eu0p, ev0p)

    u_emb, pos_emb, neg_emb, zub, zvb, evb = _gather6(
        E_u, E_v, Z_u, Z_v, uids, iids, pos, neg)

    ebp = jnp.zeros((NBP, D), jnp.float32).at[:NB].set(E_b)
    out = _losses_call(E_u, E_v, E_u_0, E_v_0, ebp, u_emb, pos_emb, neg_emb,
                       zub, zvb, evb)
    return (out[0], out[1], out[2], out[3])


# speedup vs baseline: 1.3859x; 1.0617x over previous
import functools

import jax
import jax.numpy as jnp
from jax import lax
from jax.experimental import pallas as pl
from jax.experimental.pallas import tpu as pltpu
from jax.experimental.pallas import tpu_sc as plsc

N_U = 10000
N_I = 10000
NP = 10240
E = 320000
D = 128
NB = 10
NBP = 16
B = 1024
TEMP = 0.2
L1 = 0.2
L2 = 0.2
L3 = 1e-7

NCORES = 2
NSUB = 16
NW = NCORES * NSUB

E_PER_SUB = E // NSUB
E_PER_W = E // NW
CH = 80
NCH_SUB = E_PER_SUB // CH
NCH_W = E_PER_W // CH
ROWS_SUB = NP // NSUB
ROWS_CHUNK = 32
NOUT_CH = ROWS_SUB // ROWS_CHUNK
B_PER_W = B // NW

_mesh = plsc.VectorSubcoreMesh(core_axis_name="c", subcore_axis_name="s")


def _zero_vmem(buf, nrows):
    z = jnp.zeros((16,), jnp.float32)

    def body(i, _):
        for d in range(D // 16):
            buf[i, pl.ds(d * 16, 16)] = z
        return 0

    lax.fori_loop(0, nrows, body, 0, unroll=False)


def _scale_rows(rowbuf, vbuf):
    def body(g, _):
        vg = vbuf[pl.ds(g * 16, 16)]
        for e16 in range(16):
            e = g * 16 + e16
            vj = jnp.full((16,), vg[e16], jnp.float32)
            for d in range(D // 16):
                rowbuf[e, pl.ds(d * 16, 16)] = (
                    rowbuf[e, pl.ds(d * 16, 16)] * vj)
        return 0

    lax.fori_loop(0, CH // 16, body, 0, unroll=False)


def _edge_accumulate(tab_hbm, gidx_hbm, sidx_hbm, vals_hbm, acc, slots, obuf,
                     sub):
    base = sub * E_PER_SUB
    _zero_vmem(obuf, ROWS_CHUNK)
    for p in range(NOUT_CH):
        pltpu.sync_copy(obuf, acc.at[pl.ds(sub * ROWS_SUB + p * ROWS_CHUNK,
                                           ROWS_CHUNK)])
    plsc.subcore_barrier()

    def sync_idx(j, sl):
        gidx, sidx, sctx, vbuf, rowbuf, isem, gsem, ssem = sl
        off = base + j * CH
        pltpu.sync_copy(gidx_hbm.at[pl.ds(off, CH)], gidx)
        pltpu.sync_copy(sidx_hbm.at[pl.ds(off, CH)], sidx)
        pltpu.sync_copy(vals_hbm.at[pl.ds(off, CH)], vbuf)

    def fire_idx(j, sl):
        gidx, sidx, sctx, vbuf, rowbuf, isem, gsem, ssem = sl
        off = base + j * CH
        pltpu.async_copy(gidx_hbm.at[pl.ds(off, CH)], gidx, isem)
        pltpu.async_copy(sidx_hbm.at[pl.ds(off, CH)], sidx, isem)
        pltpu.async_copy(vals_hbm.at[pl.ds(off, CH)], vbuf, isem)

    def wait_idx(sl):
        gidx, sidx, sctx, vbuf, rowbuf, isem, gsem, ssem = sl
        pltpu.make_async_copy(gidx_hbm.at[pl.ds(base, CH)], gidx, isem).wait()
        pltpu.make_async_copy(sidx_hbm.at[pl.ds(base, CH)], sidx, isem).wait()
        pltpu.make_async_copy(vals_hbm.at[pl.ds(base, CH)], vbuf, isem).wait()

    def fire_gather(sl):
        gidx, sidx, sctx, vbuf, rowbuf, isem, gsem, ssem = sl
        pltpu.async_copy(tab_hbm.at[gidx], rowbuf, gsem)

    def wait_gather(sl):
        gidx, sidx, sctx, vbuf, rowbuf, isem, gsem, ssem = sl
        pltpu.make_async_copy(tab_hbm.at[gidx], rowbuf, gsem).wait()

    def fire_scatter(sl):
        gidx, sidx, sctx, vbuf, rowbuf, isem, gsem, ssem = sl
        pltpu.async_copy(rowbuf, acc.at[sctx], ssem, add=True)

    def wait_scatter(sl):
        gidx, sidx, sctx, vbuf, rowbuf, isem, gsem, ssem = sl
        pltpu.make_async_copy(rowbuf, acc.at[sctx], ssem).wait()

    sync_idx(0, slots[0])
    sync_idx(1, slots[1])
    fire_gather(slots[0])

    def pair(jj, _):
        for b in range(2):
            j = 2 * jj + b
            sl = slots[b]
            ot = slots[1 - b]
            gidx, sidx, sctx, vbuf, rowbuf, isem, gsem, ssem = sl

            @pl.when((j >= 1) & (j + 1 < NCH_SUB))
            def _():
                wait_scatter(ot)
                wait_idx(ot)

            @pl.when(j + 1 < NCH_SUB)
            def _():
                fire_gather(ot)

            wait_gather(sl)
            for g in range(CH // 16):
                s16 = pl.ds(g * 16, 16)
                sctx[s16] = sidx[s16]
            _scale_rows(rowbuf, vbuf)
            fire_scatter(sl)

            @pl.when(j + 2 < NCH_SUB)
            def _():
                fire_idx(j + 2, sl)
        return 0

    lax.fori_loop(0, NCH_SUB // 2, pair, 0, unroll=False)
    wait_scatter(slots[0])
    wait_scatter(slots[1])
    plsc.subcore_barrier()


def _write_out_plain(acc, obuf, out_hbm, sub):
    for p in range(NOUT_CH):
        sl = pl.ds(sub * ROWS_SUB + p * ROWS_CHUNK, ROWS_CHUNK)
        pltpu.sync_copy(acc.at[sl], obuf)
        pltpu.sync_copy(obuf, out_hbm.at[sl])


def _write_out_fused(acc, obuf, bbuf, pbuf, base_hbm, prev_hbm, out_hbm, sub):
    for p in range(NOUT_CH):
        sl = pl.ds(sub * ROWS_SUB + p * ROWS_CHUNK, ROWS_CHUNK)
        pltpu.sync_copy(acc.at[sl], obuf)
        pltpu.sync_copy(base_hbm.at[sl], bbuf)
        pltpu.sync_copy(prev_hbm.at[sl], pbuf)

        def body(i, _):
            for d in range(D // 16):
                s = pl.ds(d * 16, 16)
                obuf[i, s] = obuf[i, s] + bbuf[i, s] + pbuf[i, s]
            return 0

        lax.fori_loop(0, ROWS_CHUNK, body, 0, unroll=False)
        pltpu.sync_copy(obuf, out_hbm.at[sl])


def _edge_pass_l1_body(rows_hbm, cols_hbm, vals_hbm, tabu_hbm, tabv_hbm,
                       outu_hbm, outv_hbm, acc,
                       g0, s0, c0, v0, r0, i0, gs0, ss0,
                       g1, s1, c1, v1, r1, i1, gs1, ss1, obuf):
    cid = lax.axis_index("c")
    sub = lax.axis_index("s")
    slots = ((g0, s0, c0, v0, r0, i0, gs0, ss0),
             (g1, s1, c1, v1, r1, i1, gs1, ss1))

    @pl.when(cid == 0)
    def _():
        _edge_accumulate(tabu_hbm, cols_hbm, rows_hbm, vals_hbm, acc, slots,
                         obuf, sub)
        _write_out_plain(acc, obuf, outu_hbm, sub)

    @pl.when(cid == 1)
    def _():
        _edge_accumulate(tabv_hbm, rows_hbm, cols_hbm, vals_hbm, acc, slots,
                         obuf, sub)
        _write_out_plain(acc, obuf, outv_hbm, sub)


def _edge_pass_l2_body(rows_hbm, cols_hbm, vals_hbm, tabu_hbm, tabv_hbm,
                       baseu_hbm, basev_hbm, outu_hbm, outv_hbm, acc,
                       g0, s0, c0, v0, r0, i0, gs0, ss0,
                       g1, s1, c1, v1, r1, i1, gs1, ss1, obuf, bbuf, pbuf):
    cid = lax.axis_index("c")
    sub = lax.axis_index("s")
    slots = ((g0, s0, c0, v0, r0, i0, gs0, ss0),
             (g1, s1, c1, v1, r1, i1, gs1, ss1))

    @pl.when(cid == 0)
    def _():
        _edge_accumulate(tabu_hbm, cols_hbm, rows_hbm, vals_hbm, acc, slots,
                         obuf, sub)
        _write_out_fused(acc, obuf, bbuf, pbuf, baseu_hbm, tabv_hbm, outu_hbm,
                         sub)

    @pl.when(cid == 1)
    def _():
        _edge_accumulate(tabv_hbm, rows_hbm, cols_hbm, vals_hbm, acc, slots,
                         obuf, sub)
        _write_out_fused(acc, obuf, bbuf, pbuf, basev_hbm, tabu_hbm, outv_hbm,
                         sub)


def _slot_scratch():
    return [
        pltpu.VMEM((CH,), jnp.int32),
        pltpu.VMEM((CH,), jnp.int32),
        pltpu.VMEM((CH,), jnp.int32),
        pltpu.VMEM((CH,), jnp.float32),
        pltpu.VMEM((CH, D), jnp.float32),
        pltpu.SemaphoreType.DMA,
        pltpu.SemaphoreType.DMA,
        pltpu.SemaphoreType.DMA,
    ]


_SCRATCH_COMMON = (
    [pltpu.VMEM_SHARED((NP, D), jnp.float32)]
    + _slot_scratch() + _slot_scratch()
    + [pltpu.VMEM((ROWS_CHUNK, D), jnp.float32)]
)

_edge_pass_l1 = pl.kernel(
    _edge_pass_l1_body,
    out_type=(jax.ShapeDtypeStruct((NP, D), jnp.float32),
              jax.ShapeDtypeStruct((NP, D), jnp.float32)),
    mesh=_mesh,
    scratch_types=_SCRATCH_COMMON,
    compiler_params=pltpu.CompilerParams(needs_layout_passes=False),
)

_edge_pass_l2 = pl.kernel(
    _edge_pass_l2_body,
    out_type=(jax.ShapeDtypeStruct((NP, D), jnp.float32),
              jax.ShapeDtypeStruct((NP, D), jnp.float32)),
    mesh=_mesh,
    scratch_types=_SCRATCH_COMMON + [
        pltpu.VMEM((ROWS_CHUNK, D), jnp.float32),
        pltpu.VMEM((ROWS_CHUNK, D), jnp.float32),
    ],
    compiler_params=pltpu.CompilerParams(needs_layout_passes=False),
)


def _aug_vals_body(rows_hbm, cols_hbm, adj_hbm, eu_hbm, ev_hbm, out_hbm,
                   r0, c0, a0, xu0, xi0, ob0, i0, gs0, os0,
                   r1, c1, a1, xu1, xi1, ob1, i1, gs1, os1, obig):
    wid = lax.axis_index("s") * NCORES + lax.axis_index("c")
    base = wid * E_PER_W
    slots = ((r0, c0, a0, xu0, xi0, ob0, i0, gs0, os0),
             (r1, c1, a1, xu1, xi1, ob1, i1, gs1, os1))

    def sync_idx(j, sl):
        ridx, cidx, abuf, xu, xi, obuf, isem, gsem, osem = sl
        off = base + j * CH
        pltpu.sync_copy(rows_hbm.at[pl.ds(off, CH)], ridx)
        pltpu.sync_copy(cols_hbm.at[pl.ds(off, CH)], cidx)
        pltpu.sync_copy(adj_hbm.at[pl.ds(off, CH)], abuf)

    def fire_idx(j, sl):
        ridx, cidx, abuf, xu, xi, obuf, isem, gsem, osem = sl
        off = base + j * CH
        pltpu.async_copy(rows_hbm.at[pl.ds(off, CH)], ridx, isem)
        pltpu.async_copy(cols_hbm.at[pl.ds(off, CH)], cidx, isem)
        pltpu.async_copy(adj_hbm.at[pl.ds(off, CH)], abuf, isem)

    def wait_idx(sl):
        ridx, cidx, abuf, xu, xi, obuf, isem, gsem, osem = sl
        pltpu.make_async_copy(rows_hbm.at[pl.ds(base, CH)], ridx, isem).wait()
        pltpu.make_async_copy(cols_hbm.at[pl.ds(base, CH)], cidx, isem).wait()
        pltpu.make_async_copy(adj_hbm.at[pl.ds(base, CH)], abuf, isem).wait()

    def fire_gather(sl):
        ridx, cidx, abuf, xu, xi, obuf, isem, gsem, osem = sl
        pltpu.async_copy(eu_hbm.at[ridx], xu, gsem)
        pltpu.async_copy(ev_hbm.at[cidx], xi, gsem)

    def wait_gather(sl):
        ridx, cidx, abuf, xu, xi, obuf, isem, gsem, osem = sl
        pltpu.make_async_copy(eu_hbm.at[ridx], xu, gsem).wait()
        pltpu.make_async_copy(ev_hbm.at[cidx], xi, gsem).wait()

    sync_idx(0, slots[0])
    sync_idx(1, slots[1])
    fire_gather(slots[0])
    lane = lax.iota(jnp.int32, 16)

    def pair(jj, _):
        for b in range(2):
            j = 2 * jj + b
            sl = slots[b]
            ot = slots[1 - b]
            ridx, cidx, abuf, xu, xi, obuf, isem, gsem, osem = sl

            @pl.when((j >= 1) & (j + 1 < NCH_W))
            def _():
                wait_idx(ot)

            @pl.when(j + 1 < NCH_W)
            def _():
                fire_gather(ot)

            wait_gather(sl)
            for g in range(CH // 16):
                dvec = jnp.zeros((16,), jnp.float32)
                for e16 in range(16):
                    e = g * 16 + e16
                    acc = xu[e, pl.ds(0, 16)] * xi[e, pl.ds(0, 16)]
                    for d in range(1, D // 16):
                        s = pl.ds(d * 16, 16)
                        acc = acc + xu[e, s] * xi[e, s]
                    for k in (8, 4, 2, 1):
                        acc = acc + jnp.take(acc, lane ^ k)
                    dvec = jnp.where(lane == e16, acc, dvec)
                obig[pl.ds((j % 5) * CH + g * 16, 16)] = (
                    abuf[pl.ds(g * 16, 16)] / (1.0 + jnp.exp(-dvec)))

            @pl.when(j % 5 == 4)
            def _():
                pltpu.sync_copy(
                    obig, out_hbm.at[pl.ds(base + (j - 4) * CH, 5 * CH)])

            @pl.when(j + 2 < NCH_W)
            def _():
                fire_idx(j + 2, sl)
        return 0

    lax.fori_loop(0, NCH_W // 2, pair, 0, unroll=False)


def _aug_slot_scratch():
    return [
        pltpu.VMEM((CH,), jnp.int32),
        pltpu.VMEM((CH,), jnp.int32),
        pltpu.VMEM((CH,), jnp.float32),
        pltpu.VMEM((CH, D), jnp.float32),
        pltpu.VMEM((CH, D), jnp.float32),
        pltpu.VMEM((CH,), jnp.float32),
        pltpu.SemaphoreType.DMA,
        pltpu.SemaphoreType.DMA,
        pltpu.SemaphoreType.DMA,
    ]


_aug_vals = pl.kernel(
    _aug_vals_body,
    out_type=jax.ShapeDtypeStruct((E,), jnp.float32),
    mesh=_mesh,
    scratch_types=(_aug_slot_scratch() + _aug_slot_scratch()
                   + [pltpu.VMEM((5 * CH,), jnp.float32)]),
    compiler_params=pltpu.CompilerParams(needs_layout_passes=False),
)


def _gather6_body(eu_hbm, ev_hbm, zu_hbm, zv_hbm, uids_hbm, iids_hbm, pos_hbm,
                  neg_hbm, o_uemb, o_pos, o_neg, o_zub, o_zvb, o_evb,
                  ibuf, rbuf):
    wid = lax.axis_index("s") * NCORES + lax.axis_index("c")
    sl = pl.ds(wid * B_PER_W, B_PER_W)
    for idx_hbm, tab_hbm, out_hbm in (
        (uids_hbm, eu_hbm, o_uemb),
        (pos_hbm, ev_hbm, o_pos),
        (neg_hbm, ev_hbm, o_neg),
        (uids_hbm, zu_hbm, o_zub),
        (iids_hbm, zv_hbm, o_zvb),
        (iids_hbm, ev_hbm, o_evb),
    ):
        pltpu.sync_copy(idx_hbm.at[sl], ibuf)
        pltpu.sync_copy(tab_hbm.at[ibuf], rbuf)
        pltpu.sync_copy(rbuf, out_hbm.at[sl])


_gather6 = pl.kernel(
    _gather6_body,
    out_type=tuple(jax.ShapeDtypeStruct((B, D), jnp.float32)
                   for _ in range(6)),
    mesh=_mesh,
    scratch_types=[
        pltpu.VMEM((B_PER_W,), jnp.int32),
        pltpu.VMEM((B_PER_W, D), jnp.float32),
    ],
    compiler_params=pltpu.CompilerParams(needs_layout_passes=False),
)


def _losses_body(eu_ref, ev_ref, eu0_ref, ev0_ref, ebp_ref, uemb_ref, pos_ref,
                 neg_ref, zub_ref, zvb_ref, evb_ref, out_ref):
    u_emb = uemb_ref[...]
    pos_emb = pos_ref[...]
    neg_emb = neg_ref[...]
    zub = zub_ref[...]
    zvb = zvb_ref[...]
    evb = evb_ref[...]

    pos_scores = jnp.sum(u_emb * pos_emb, axis=1, keepdims=True)
    neg_scores = jnp.sum(u_emb * neg_emb, axis=1, keepdims=True)
    diff = pos_scores - neg_scores
    sig = 1.0 / (1.0 + jnp.exp(-diff))
    loss_bpr = -jnp.sum(jnp.log(sig)) / B

    def pcl_neg(zb, tab_ref):
        def blk(k, acc):
            t = tab_ref[pl.ds(k * 1000, 1000), :]
            s = lax.dot_general(zb, t, (((1,), (1,)), ((), ())),
                                preferred_element_type=jnp.float32)
            return acc + jnp.sum(jnp.exp(s / TEMP), axis=1, keepdims=True)

        acc = lax.fori_loop(0, N_U // 1000, blk,
                            jnp.zeros((B, 1), jnp.float32))
        return jnp.sum(jnp.log(acc + 1e-8)) / B

    neg_s = pcl_neg(zub, eu_ref) + pcl_neg(zvb, ev_ref)
    pos_s = (jnp.sum(jnp.clip(jnp.sum(zub * u_emb, axis=1) / TEMP, -5.0, 5.0))
             / B
             + jnp.sum(jnp.clip(jnp.sum(zvb * evb, axis=1) / TEMP, -5.0, 5.0))
             / B)
    loss_pcl = -pos_s + neg_s

    ps_min = jnp.min(pos_scores)
    ps_max = jnp.max(pos_scores)
    weight_b = (pos_scores - ps_min) / (ps_max - ps_min + 1e-9)
    relations = jnp.clip((weight_b * NB).astype(jnp.int32), 0, NB - 1)
    el = 1.0 / (1.0 + jnp.exp(-(u_emb * pos_emb)))
    s_all = lax.dot_general(el, ebp_ref[...], (((1,), (1,)), ((), ())),
                            preferred_element_type=jnp.float32)
    lane = lax.broadcasted_iota(jnp.int32, (B, NBP), 1)
    onehot = lane == relations
    srel = jnp.sum(jnp.where(onehot, s_all, 0.0), axis=1, keepdims=True)
    ssum = jnp.sum(s_all, axis=1, keepdims=True)
    neg_bcl = jnp.sum((ssum - srel) / NB) / B
    pos_bcl = jnp.sum(srel) / B
    loss_bcl = neg_bcl - pos_bcl

    def sq(tab_ref):
        def blk(k, acc):
            t = tab_ref[pl.ds(k * 200, 200), :]
            return acc + jnp.sum(t * t)

        return lax.fori_loop(0, N_U // 200, blk, jnp.float32(0.0))

    loss_reg = L3 * (sq(eu0_ref) + sq(ev0_ref) + jnp.sum(ebp_ref[...] ** 2))

    loss = loss_bpr + L1 * loss_pcl + L2 * loss_bcl + loss_reg
    out_ref[0] = loss
    out_ref[1] = loss_bpr
    out_ref[2] = L1 * loss_pcl
    out_ref[3] = L2 * loss_bcl


def _losses_call(eu, ev, eu0, ev0, ebp, uemb, posb, negb, zub, zvb, evb):
    return pl.pallas_call(
        _losses_body,
        out_shape=jax.ShapeDtypeStruct((4,), jnp.float32),
        in_specs=[pl.BlockSpec(memory_space=pltpu.VMEM)] * 11,
        out_specs=pl.BlockSpec(memory_space=pltpu.SMEM),
    )(eu, ev, eu0, ev0, ebp, uemb, posb, negb, zub, zvb, evb)


def kernel(E_u_0, E_v_0, E_b, adj_vals, edgE_vndex, uids, iids, pos, neg):
    rows = edgE_vndex[0]
    cols = edgE_vndex[1]
    pad = ((0, NP - N_U), (0, 0))
    eu0p = jnp.pad(E_u_0, pad)
    ev0p = jnp.pad(E_v_0, pad)

    nu1, nv1 = _edge_pass_l1(rows, cols, adj_vals, ev0p, eu0p)
    E_u, E_v = _edge_pass_l2(rows, cols, adj_vals, nv1, nu1, eu0p, ev0p)

    aug = _aug_vals(rows, cols, adj_vals, E_u, E_v)

    m_u1, m_v1 = _edge_pass_l1(rows, cols, aug, ev0p, eu0p)
    Z_u, Z_v = _edge_pass_l2(rows, cols, aug, m_v1, m_u1, eu0p, ev0p)

    u_emb, pos_emb, neg_emb, zub, zvb, evb = _gather6(
        E_u, E_v, Z_u, Z_v, uids, iids, pos, neg)

    ebp = jnp.zeros((NBP, D), jnp.float32).at[:NB].set(E_b)
    out = _losses_call(E_u, E_v, E_u_0, E_v_0, ebp, u_emb, pos_emb, neg_emb,
                       zub, zvb, evb)
    return (out[0], out[1], out[2], out[3])


# speedup vs baseline: 1.6810x; 1.2129x over previous
import functools

import jax
import jax.numpy as jnp
from jax import lax
from jax.experimental import pallas as pl
from jax.experimental.pallas import tpu as pltpu
from jax.experimental.pallas import tpu_sc as plsc

N_U = 10000
N_I = 10000
NP = 10240
E = 320000
D = 128
NB = 10
NBP = 16
B = 1024
TEMP = 0.2
L1 = 0.2
L2 = 0.2
L3 = 1e-7

NCORES = 2
NSUB = 16
NW = NCORES * NSUB

E_PER_SUB = E // NSUB
E_PER_W = E // NW
CH = 80
NCH_SUB = E_PER_SUB // CH
NCH_W = E_PER_W // CH
ROWS_SUB = NP // NSUB
ROWS_CHUNK = 32
NOUT_CH = ROWS_SUB // ROWS_CHUNK
B_PER_W = B // NW

_mesh = plsc.VectorSubcoreMesh(core_axis_name="c", subcore_axis_name="s")


def _zero_vmem(buf, nrows):
    z = jnp.zeros((16,), jnp.float32)

    def body(i, _):
        for d in range(D // 16):
            buf[i, pl.ds(d * 16, 16)] = z
        return 0

    lax.fori_loop(0, nrows, body, 0, unroll=False)


def _scale_rows(rowbuf, vbuf):
    def body(g, _):
        vg = vbuf[pl.ds(g * 16, 16)]
        for e16 in range(16):
            e = g * 16 + e16
            vj = jnp.full((16,), vg[e16], jnp.float32)
            for d in range(D // 16):
                rowbuf[e, pl.ds(d * 16, 16)] = (
                    rowbuf[e, pl.ds(d * 16, 16)] * vj)
        return 0

    lax.fori_loop(0, CH // 16, body, 0, unroll=False)


def _edge_accumulate(tab_hbm, gidx_hbm, sidx_hbm, vals_hbm, acc, slots, obuf,
                     sub):
    base = sub * E_PER_SUB
    _zero_vmem(obuf, ROWS_CHUNK)
    for p in range(NOUT_CH):
        pltpu.sync_copy(obuf, acc.at[pl.ds(sub * ROWS_SUB + p * ROWS_CHUNK,
                                           ROWS_CHUNK)])
    plsc.subcore_barrier()

    def sync_idx(j, sl):
        gidx, sidx, sctx, vbuf, rowbuf, isem, gsem, ssem = sl
        off = base + j * CH
        pltpu.sync_copy(gidx_hbm.at[pl.ds(off, CH)], gidx)
        pltpu.sync_copy(sidx_hbm.at[pl.ds(off, CH)], sidx)
        pltpu.sync_copy(vals_hbm.at[pl.ds(off, CH)], vbuf)

    def fire_idx(j, sl):
        gidx, sidx, sctx, vbuf, rowbuf, isem, gsem, ssem = sl
        off = base + j * CH
        pltpu.async_copy(gidx_hbm.at[pl.ds(off, CH)], gidx, isem)
        pltpu.async_copy(sidx_hbm.at[pl.ds(off, CH)], sidx, isem)
        pltpu.async_copy(vals_hbm.at[pl.ds(off, CH)], vbuf, isem)

    def wait_idx(sl):
        gidx, sidx, sctx, vbuf, rowbuf, isem, gsem, ssem = sl
        pltpu.make_async_copy(gidx_hbm.at[pl.ds(base, CH)], gidx, isem).wait()
        pltpu.make_async_copy(sidx_hbm.at[pl.ds(base, CH)], sidx, isem).wait()
        pltpu.make_async_copy(vals_hbm.at[pl.ds(base, CH)], vbuf, isem).wait()

    def fire_gather(sl):
        gidx, sidx, sctx, vbuf, rowbuf, isem, gsem, ssem = sl
        pltpu.async_copy(tab_hbm.at[gidx], rowbuf, gsem)

    def wait_gather(sl):
        gidx, sidx, sctx, vbuf, rowbuf, isem, gsem, ssem = sl
        pltpu.make_async_copy(tab_hbm.at[gidx], rowbuf, gsem).wait()

    def fire_scatter(sl):
        gidx, sidx, sctx, vbuf, rowbuf, isem, gsem, ssem = sl
        pltpu.async_copy(rowbuf, acc.at[sctx], ssem, add=True)

    def wait_scatter(sl):
        gidx, sidx, sctx, vbuf, rowbuf, isem, gsem, ssem = sl
        pltpu.make_async_copy(rowbuf, acc.at[sctx], ssem).wait()

    sync_idx(0, slots[0])
    sync_idx(1, slots[1])
    fire_gather(slots[0])

    def pair(jj, _):
        for b in range(2):
            j = 2 * jj + b
            sl = slots[b]
            ot = slots[1 - b]
            gidx, sidx, sctx, vbuf, rowbuf, isem, gsem, ssem = sl

            @pl.when((j >= 1) & (j + 1 < NCH_SUB))
            def _():
                wait_scatter(ot)
                wait_idx(ot)

            @pl.when(j + 1 < NCH_SUB)
            def _():
                fire_gather(ot)

            wait_gather(sl)
            for g in range(CH // 16):
                s16 = pl.ds(g * 16, 16)
                sctx[s16] = sidx[s16]
            _scale_rows(rowbuf, vbuf)
            fire_scatter(sl)

            @pl.when(j + 2 < NCH_SUB)
            def _():
                fire_idx(j + 2, sl)
        return 0

    lax.fori_loop(0, NCH_SUB // 2, pair, 0, unroll=False)
    wait_scatter(slots[0])
    wait_scatter(slots[1])
    plsc.subcore_barrier()


def _write_out_plain(acc, obuf, out_hbm, sub):
    for p in range(NOUT_CH):
        sl = pl.ds(sub * ROWS_SUB + p * ROWS_CHUNK, ROWS_CHUNK)
        pltpu.sync_copy(acc.at[sl], obuf)
        pltpu.sync_copy(obuf, out_hbm.at[sl])


def _write_out_fused(acc, obuf, bbuf, pbuf, base_hbm, prev_hbm, out_hbm, sub):
    for p in range(NOUT_CH):
        sl = pl.ds(sub * ROWS_SUB + p * ROWS_CHUNK, ROWS_CHUNK)
        pltpu.sync_copy(acc.at[sl], obuf)
        pltpu.sync_copy(base_hbm.at[sl], bbuf)
        pltpu.sync_copy(prev_hbm.at[sl], pbuf)

        def body(i, _):
            for d in range(D // 16):
                s = pl.ds(d * 16, 16)
                obuf[i, s] = obuf[i, s] + bbuf[i, s] + pbuf[i, s]
            return 0

        lax.fori_loop(0, ROWS_CHUNK, body, 0, unroll=False)
        pltpu.sync_copy(obuf, out_hbm.at[sl])


def _edge_pass_l1_body(rows_hbm, cols_hbm, vals_hbm, tabu_hbm, tabv_hbm,
                       outu_hbm, outv_hbm, acc,
                       g0, s0, c0, v0, r0, i0, gs0, ss0,
                       g1, s1, c1, v1, r1, i1, gs1, ss1, obuf):
    cid = lax.axis_index("c")
    sub = lax.axis_index("s")
    slots = ((g0, s0, c0, v0, r0, i0, gs0, ss0),
             (g1, s1, c1, v1, r1, i1, gs1, ss1))

    @pl.when(cid == 0)
    def _():
        _edge_accumulate(tabu_hbm, cols_hbm, rows_hbm, vals_hbm, acc, slots,
                         obuf, sub)
        _write_out_plain(acc, obuf, outu_hbm, sub)

    @pl.when(cid == 1)
    def _():
        _edge_accumulate(tabv_hbm, rows_hbm, cols_hbm, vals_hbm, acc, slots,
                         obuf, sub)
        _write_out_plain(acc, obuf, outv_hbm, sub)


def _edge_pass_l2_body(rows_hbm, cols_hbm, vals_hbm, tabu_hbm, tabv_hbm,
                       baseu_hbm, basev_hbm, outu_hbm, outv_hbm, acc,
                       g0, s0, c0, v0, r0, i0, gs0, ss0,
                       g1, s1, c1, v1, r1, i1, gs1, ss1, obuf, bbuf, pbuf):
    cid = lax.axis_index("c")
    sub = lax.axis_index("s")
    slots = ((g0, s0, c0, v0, r0, i0, gs0, ss0),
             (g1, s1, c1, v1, r1, i1, gs1, ss1))

    @pl.when(cid == 0)
    def _():
        _edge_accumulate(tabu_hbm, cols_hbm, rows_hbm, vals_hbm, acc, slots,
                         obuf, sub)
        _write_out_fused(acc, obuf, bbuf, pbuf, baseu_hbm, tabv_hbm, outu_hbm,
                         sub)

    @pl.when(cid == 1)
    def _():
        _edge_accumulate(tabv_hbm, rows_hbm, cols_hbm, vals_hbm, acc, slots,
                         obuf, sub)
        _write_out_fused(acc, obuf, bbuf, pbuf, basev_hbm, tabu_hbm, outv_hbm,
                         sub)


def _slot_scratch():
    return [
        pltpu.VMEM((CH,), jnp.int32),
        pltpu.VMEM((CH,), jnp.int32),
        pltpu.VMEM((CH,), jnp.int32),
        pltpu.VMEM((CH,), jnp.float32),
        pltpu.VMEM((CH, D), jnp.float32),
        pltpu.SemaphoreType.DMA,
        pltpu.SemaphoreType.DMA,
        pltpu.SemaphoreType.DMA,
    ]


_SCRATCH_COMMON = (
    [pltpu.VMEM_SHARED((NP, D), jnp.float32)]
    + _slot_scratch() + _slot_scratch()
    + [pltpu.VMEM((ROWS_CHUNK, D), jnp.float32)]
)

_edge_pass_l1 = pl.kernel(
    _edge_pass_l1_body,
    out_type=(jax.ShapeDtypeStruct((NP, D), jnp.float32),
              jax.ShapeDtypeStruct((NP, D), jnp.float32)),
    mesh=_mesh,
    scratch_types=_SCRATCH_COMMON,
    compiler_params=pltpu.CompilerParams(needs_layout_passes=False),
)

_edge_pass_l2 = pl.kernel(
    _edge_pass_l2_body,
    out_type=(jax.ShapeDtypeStruct((NP, D), jnp.float32),
              jax.ShapeDtypeStruct((NP, D), jnp.float32)),
    mesh=_mesh,
    scratch_types=_SCRATCH_COMMON + [
        pltpu.VMEM((ROWS_CHUNK, D), jnp.float32),
        pltpu.VMEM((ROWS_CHUNK, D), jnp.float32),
    ],
    compiler_params=pltpu.CompilerParams(needs_layout_passes=False),
)


def _aug_vals_body(rows_hbm, cols_hbm, adj_hbm, eu_hbm, ev_hbm, out_hbm,
                   r0, c0, a0, xu0, xi0, ob0, i0, gs0, os0,
                   r1, c1, a1, xu1, xi1, ob1, i1, gs1, os1, obig):
    wid = lax.axis_index("s") * NCORES + lax.axis_index("c")
    base = wid * E_PER_W
    slots = ((r0, c0, a0, xu0, xi0, ob0, i0, gs0, os0),
             (r1, c1, a1, xu1, xi1, ob1, i1, gs1, os1))

    def sync_idx(j, sl):
        ridx, cidx, abuf, xu, xi, obuf, isem, gsem, osem = sl
        off = base + j * CH
        pltpu.sync_copy(rows_hbm.at[pl.ds(off, CH)], ridx)
        pltpu.sync_copy(cols_hbm.at[pl.ds(off, CH)], cidx)
        pltpu.sync_copy(adj_hbm.at[pl.ds(off, CH)], abuf)

    def fire_idx(j, sl):
        ridx, cidx, abuf, xu, xi, obuf, isem, gsem, osem = sl
        off = base + j * CH
        pltpu.async_copy(rows_hbm.at[pl.ds(off, CH)], ridx, isem)
        pltpu.async_copy(cols_hbm.at[pl.ds(off, CH)], cidx, isem)
        pltpu.async_copy(adj_hbm.at[pl.ds(off, CH)], abuf, isem)

    def wait_idx(sl):
        ridx, cidx, abuf, xu, xi, obuf, isem, gsem, osem = sl
        pltpu.make_async_copy(rows_hbm.at[pl.ds(base, CH)], ridx, isem).wait()
        pltpu.make_async_copy(cols_hbm.at[pl.ds(base, CH)], cidx, isem).wait()
        pltpu.make_async_copy(adj_hbm.at[pl.ds(base, CH)], abuf, isem).wait()

    def fire_gather(sl):
        ridx, cidx, abuf, xu, xi, obuf, isem, gsem, osem = sl
        pltpu.async_copy(eu_hbm.at[ridx], xu, gsem)
        pltpu.async_copy(ev_hbm.at[cidx], xi, gsem)

    def wait_gather(sl):
        ridx, cidx, abuf, xu, xi, obuf, isem, gsem, osem = sl
        pltpu.make_async_copy(eu_hbm.at[ridx], xu, gsem).wait()
        pltpu.make_async_copy(ev_hbm.at[cidx], xi, gsem).wait()

    sync_idx(0, slots[0])
    sync_idx(1, slots[1])
    fire_gather(slots[0])
    lane = lax.iota(jnp.int32, 16)

    def pair(jj, _):
        for b in range(2):
            j = 2 * jj + b
            sl = slots[b]
            ot = slots[1 - b]
            ridx, cidx, abuf, xu, xi, obuf, isem, gsem, osem = sl

            @pl.when((j >= 1) & (j + 1 < NCH_W))
            def _():
                wait_idx(ot)

            @pl.when(j + 1 < NCH_W)
            def _():
                fire_gather(ot)

            wait_gather(sl)

            def group(g, _):
                def edge_fn(e16, dvec):
                    e = g * 16 + e16
                    acc = xu[e, pl.ds(0, 16)] * xi[e, pl.ds(0, 16)]
                    for d in range(1, D // 16):
                        s = pl.ds(d * 16, 16)
                        acc = acc + xu[e, s] * xi[e, s]
                    for k in (8, 4, 2, 1):
                        acc = acc + jnp.take(acc, lane ^ k)
                    return jnp.where(lane == e16, acc, dvec)

                dvec = lax.fori_loop(0, 16, edge_fn,
                                     jnp.zeros((16,), jnp.float32))
                obig[pl.ds((j % 5) * CH + g * 16, 16)] = (
                    abuf[pl.ds(g * 16, 16)] / (1.0 + jnp.exp(-dvec)))
                return 0

            lax.fori_loop(0, CH // 16, group, 0, unroll=False)

            @pl.when(j % 5 == 4)
            def _():
                pltpu.sync_copy(
                    obig, out_hbm.at[pl.ds(base + (j - 4) * CH, 5 * CH)])

            @pl.when(j + 2 < NCH_W)
            def _():
                fire_idx(j + 2, sl)
        return 0

    lax.fori_loop(0, NCH_W // 2, pair, 0, unroll=False)


def _aug_slot_scratch():
    return [
        pltpu.VMEM((CH,), jnp.int32),
        pltpu.VMEM((CH,), jnp.int32),
        pltpu.VMEM((CH,), jnp.float32),
        pltpu.VMEM((CH, D), jnp.float32),
        pltpu.VMEM((CH, D), jnp.float32),
        pltpu.VMEM((CH,), jnp.float32),
        pltpu.SemaphoreType.DMA,
        pltpu.SemaphoreType.DMA,
        pltpu.SemaphoreType.DMA,
    ]


_aug_vals = pl.kernel(
    _aug_vals_body,
    out_type=jax.ShapeDtypeStruct((E,), jnp.float32),
    mesh=_mesh,
    scratch_types=(_aug_slot_scratch() + _aug_slot_scratch()
                   + [pltpu.VMEM((5 * CH,), jnp.float32)]),
    compiler_params=pltpu.CompilerParams(needs_layout_passes=False),
)


def _gather6_body(eu_hbm, ev_hbm, zu_hbm, zv_hbm, uids_hbm, iids_hbm, pos_hbm,
                  neg_hbm, o_uemb, o_pos, o_neg, o_zub, o_zvb, o_evb,
                  ibuf, rbuf):
    wid = lax.axis_index("s") * NCORES + lax.axis_index("c")
    sl = pl.ds(wid * B_PER_W, B_PER_W)
    for idx_hbm, tab_hbm, out_hbm in (
        (uids_hbm, eu_hbm, o_uemb),
        (pos_hbm, ev_hbm, o_pos),
        (neg_hbm, ev_hbm, o_neg),
        (uids_hbm, zu_hbm, o_zub),
        (iids_hbm, zv_hbm, o_zvb),
        (iids_hbm, ev_hbm, o_evb),
    ):
        pltpu.sync_copy(idx_hbm.at[sl], ibuf)
        pltpu.sync_copy(tab_hbm.at[ibuf], rbuf)
        pltpu.sync_copy(rbuf, out_hbm.at[sl])


_gather6 = pl.kernel(
    _gather6_body,
    out_type=tuple(jax.ShapeDtypeStruct((B, D), jnp.float32)
                   for _ in range(6)),
    mesh=_mesh,
    scratch_types=[
        pltpu.VMEM((B_PER_W,), jnp.int32),
        pltpu.VMEM((B_PER_W, D), jnp.float32),
    ],
    compiler_params=pltpu.CompilerParams(needs_layout_passes=False),
)


def _losses_body(eu_ref, ev_ref, eu0_ref, ev0_ref, ebp_ref, uemb_ref, pos_ref,
                 neg_ref, zub_ref, zvb_ref, evb_ref, out_ref):
    u_emb = uemb_ref[...]
    pos_emb = pos_ref[...]
    neg_emb = neg_ref[...]
    zub = zub_ref[...]
    zvb = zvb_ref[...]
    evb = evb_ref[...]

    pos_scores = jnp.sum(u_emb * pos_emb, axis=1, keepdims=True)
    neg_scores = jnp.sum(u_emb * neg_emb, axis=1, keepdims=True)
    diff = pos_scores - neg_scores
    sig = 1.0 / (1.0 + jnp.exp(-diff))
    loss_bpr = -jnp.sum(jnp.log(sig)) / B

    def pcl_neg(zb, tab_ref):
        def blk(k, acc):
            t = tab_ref[pl.ds(k * 1000, 1000), :]
            s = lax.dot_general(zb, t, (((1,), (1,)), ((), ())),
                                preferred_element_type=jnp.float32)
            return acc + jnp.sum(jnp.exp(s / TEMP), axis=1, keepdims=True)

        acc = lax.fori_loop(0, N_U // 1000, blk,
                            jnp.zeros((B, 1), jnp.float32))
        return jnp.sum(jnp.log(acc + 1e-8)) / B

    neg_s = pcl_neg(zub, eu_ref) + pcl_neg(zvb, ev_ref)
    pos_s = (jnp.sum(jnp.clip(jnp.sum(zub * u_emb, axis=1) / TEMP, -5.0, 5.0))
             / B
             + jnp.sum(jnp.clip(jnp.sum(zvb * evb, axis=1) / TEMP, -5.0, 5.0))
             / B)
    loss_pcl = -pos_s + neg_s

    ps_min = jnp.min(pos_scores)
    ps_max = jnp.max(pos_scores)
    weight_b = (pos_scores - ps_min) / (ps_max - ps_min + 1e-9)
    relations = jnp.clip((weight_b * NB).astype(jnp.int32), 0, NB - 1)
    el = 1.0 / (1.0 + jnp.exp(-(u_emb * pos_emb)))
    s_all = lax.dot_general(el, ebp_ref[...], (((1,), (1,)), ((), ())),
                            preferred_element_type=jnp.float32)
    lane = lax.broadcasted_iota(jnp.int32, (B, NBP), 1)
    onehot = lane == relations
    srel = jnp.sum(jnp.where(onehot, s_all, 0.0), axis=1, keepdims=True)
    ssum = jnp.sum(s_all, axis=1, keepdims=True)
    neg_bcl = jnp.sum((ssum - srel) / NB) / B
    pos_bcl = jnp.sum(srel) / B
    loss_bcl = neg_bcl - pos_bcl

    def sq(tab_ref):
        def blk(k, acc):
            t = tab_ref[pl.ds(k * 200, 200), :]
            return acc + jnp.sum(t * t)

        return lax.fori_loop(0, N_U // 200, blk, jnp.float32(0.0))

    loss_reg = L3 * (sq(eu0_ref) + sq(ev0_ref) + jnp.sum(ebp_ref[...] ** 2))

    loss = loss_bpr + L1 * loss_pcl + L2 * loss_bcl + loss_reg
    out_ref[0] = loss
    out_ref[1] = loss_bpr
    out_ref[2] = L1 * loss_pcl
    out_ref[3] = L2 * loss_bcl


def _losses_call(eu, ev, eu0, ev0, ebp, uemb, posb, negb, zub, zvb, evb):
    return pl.pallas_call(
        _losses_body,
        out_shape=jax.ShapeDtypeStruct((4,), jnp.float32),
        in_specs=[pl.BlockSpec(memory_space=pltpu.VMEM)] * 11,
        out_specs=pl.BlockSpec(memory_space=pltpu.SMEM),
    )(eu, ev, eu0, ev0, ebp, uemb, posb, negb, zub, zvb, evb)


def kernel(E_u_0, E_v_0, E_b, adj_vals, edgE_vndex, uids, iids, pos, neg):
    rows = edgE_vndex[0]
    cols = edgE_vndex[1]
    pad = ((0, NP - N_U), (0, 0))
    eu0p = jnp.pad(E_u_0, pad)
    ev0p = jnp.pad(E_v_0, pad)

    nu1, nv1 = _edge_pass_l1(rows, cols, adj_vals, ev0p, eu0p)
    E_u, E_v = _edge_pass_l2(rows, cols, adj_vals, nv1, nu1, eu0p, ev0p)

    aug = _aug_vals(rows, cols, adj_vals, E_u, E_v)

    m_u1, m_v1 = _edge_pass_l1(rows, cols, aug, ev0p, eu0p)
    Z_u, Z_v = _edge_pass_l2(rows, cols, aug, m_v1, m_u1, eu0p, ev0p)

    u_emb, pos_emb, neg_emb, zub, zvb, evb = _gather6(
        E_u, E_v, Z_u, Z_v, uids, iids, pos, neg)

    ebp = jnp.zeros((NBP, D), jnp.float32).at[:NB].set(E_b)
    out = _losses_call(E_u, E_v, E_u_0, E_v_0, ebp, u_emb, pos_emb, neg_emb,
                       zub, zvb, evb)
    return (out[0], out[1], out[2], out[3])


# speedup vs baseline: 1.7125x; 1.0187x over previous
import functools

import jax
import jax.numpy as jnp
from jax import lax
from jax.experimental import pallas as pl
from jax.experimental.pallas import tpu as pltpu
from jax.experimental.pallas import tpu_sc as plsc

N_U = 10000
N_I = 10000
NP = 10240
E = 320000
D = 128
NB = 10
NBP = 16
B = 1024
TEMP = 0.2
L1 = 0.2
L2 = 0.2
L3 = 1e-7

NCORES = 2
NSUB = 16
NW = NCORES * NSUB

E_PER_SUB = E // NSUB
E_PER_W = E // NW
CH = 80
NCH_SUB = E_PER_SUB // CH
NCH_W = E_PER_W // CH
ROWS_SUB = NP // NSUB
ROWS_CHUNK = 32
NOUT_CH = ROWS_SUB // ROWS_CHUNK
B_PER_W = B // NW

_mesh = plsc.VectorSubcoreMesh(core_axis_name="c", subcore_axis_name="s")


def _zero_vmem(buf, nrows):
    z = jnp.zeros((16,), jnp.float32)

    def body(i, _):
        for d in range(D // 16):
            buf[i, pl.ds(d * 16, 16)] = z
        return 0

    lax.fori_loop(0, nrows, body, 0, unroll=False)


def _scale_rows(rowbuf, vbuf):
    def body(g, _):
        vg = vbuf[pl.ds(g * 16, 16)]
        for e16 in range(16):
            e = g * 16 + e16
            vj = jnp.full((16,), vg[e16], jnp.float32)
            for d in range(D // 16):
                rowbuf[e, pl.ds(d * 16, 16)] = (
                    rowbuf[e, pl.ds(d * 16, 16)] * vj)
        return 0

    lax.fori_loop(0, CH // 16, body, 0, unroll=False)


def _edge_accumulate(tab_hbm, gidx_hbm, sidx_hbm, vals_hbm, acc, slots, obuf,
                     sub):
    base = sub * E_PER_SUB
    _zero_vmem(obuf, ROWS_CHUNK)
    for p in range(NOUT_CH):
        pltpu.sync_copy(obuf, acc.at[pl.ds(sub * ROWS_SUB + p * ROWS_CHUNK,
                                           ROWS_CHUNK)])
    plsc.subcore_barrier()

    def sync_idx(j, sl):
        gidx, sidx, sctx, vbuf, rowbuf, isem, gsem, ssem = sl
        off = base + j * CH
        pltpu.sync_copy(gidx_hbm.at[pl.ds(off, CH)], gidx)
        pltpu.sync_copy(sidx_hbm.at[pl.ds(off, CH)], sidx)
        pltpu.sync_copy(vals_hbm.at[pl.ds(off, CH)], vbuf)

    def fire_idx(j, sl):
        gidx, sidx, sctx, vbuf, rowbuf, isem, gsem, ssem = sl
        off = base + j * CH
        pltpu.async_copy(gidx_hbm.at[pl.ds(off, CH)], gidx, isem)
        pltpu.async_copy(sidx_hbm.at[pl.ds(off, CH)], sidx, isem)
        pltpu.async_copy(vals_hbm.at[pl.ds(off, CH)], vbuf, isem)

    def wait_idx(sl):
        gidx, sidx, sctx, vbuf, rowbuf, isem, gsem, ssem = sl
        pltpu.make_async_copy(gidx_hbm.at[pl.ds(base, CH)], gidx, isem).wait()
        pltpu.make_async_copy(sidx_hbm.at[pl.ds(base, CH)], sidx, isem).wait()
        pltpu.make_async_copy(vals_hbm.at[pl.ds(base, CH)], vbuf, isem).wait()

    def fire_gather(sl):
        gidx, sidx, sctx, vbuf, rowbuf, isem, gsem, ssem = sl
        pltpu.async_copy(tab_hbm.at[gidx], rowbuf, gsem)

    def wait_gather(sl):
        gidx, sidx, sctx, vbuf, rowbuf, isem, gsem, ssem = sl
        pltpu.make_async_copy(tab_hbm.at[gidx], rowbuf, gsem).wait()

    def fire_scatter(sl):
        gidx, sidx, sctx, vbuf, rowbuf, isem, gsem, ssem = sl
        pltpu.async_copy(rowbuf, acc.at[sctx], ssem, add=True)

    def wait_scatter(sl):
        gidx, sidx, sctx, vbuf, rowbuf, isem, gsem, ssem = sl
        pltpu.make_async_copy(rowbuf, acc.at[sctx], ssem).wait()

    NPIPE = NCH_SUB - 1
    sync_idx(0, slots[0])
    sync_idx(1, slots[1])
    sync_idx(2, slots[2])
    fire_gather(slots[0])
    fire_gather(slots[1])

    def triple(tt, _):
        for b in range(3):
            j = 3 * tt + b
            sl = slots[b]
            m = slots[(b + 2) % 3]
            gidx, sidx, sctx, vbuf, rowbuf, isem, gsem, ssem = sl

            @pl.when((j >= 1) & (j + 2 < NPIPE))
            def _():
                wait_scatter(m)
                wait_idx(m)

            @pl.when(j + 2 < NPIPE)
            def _():
                fire_gather(m)

            wait_gather(sl)
            for g in range(CH // 16):
                s16 = pl.ds(g * 16, 16)
                sctx[s16] = sidx[s16]
            _scale_rows(rowbuf, vbuf)
            fire_scatter(sl)

            @pl.when(j + 3 < NPIPE)
            def _():
                fire_idx(j + 3, sl)
        return 0

    lax.fori_loop(0, NPIPE // 3, triple, 0, unroll=False)
    wait_scatter(slots[0])
    wait_scatter(slots[1])
    wait_scatter(slots[2])
    sync_idx(NCH_SUB - 1, slots[0])
    fire_gather(slots[0])
    wait_gather(slots[0])
    g0, si0, sc0, vb0, rb0 = slots[0][:5]
    for g in range(CH // 16):
        s16 = pl.ds(g * 16, 16)
        sc0[s16] = si0[s16]
    _scale_rows(rb0, vb0)
    fire_scatter(slots[0])
    wait_scatter(slots[0])
    plsc.subcore_barrier()


def _write_out_plain(acc, obuf, out_hbm, sub):
    for p in range(NOUT_CH):
        sl = pl.ds(sub * ROWS_SUB + p * ROWS_CHUNK, ROWS_CHUNK)
        pltpu.sync_copy(acc.at[sl], obuf)
        pltpu.sync_copy(obuf, out_hbm.at[sl])


def _write_out_fused(acc, obuf, bbuf, pbuf, base_hbm, prev_hbm, out_hbm, sub):
    for p in range(NOUT_CH):
        sl = pl.ds(sub * ROWS_SUB + p * ROWS_CHUNK, ROWS_CHUNK)
        pltpu.sync_copy(acc.at[sl], obuf)
        pltpu.sync_copy(base_hbm.at[sl], bbuf)
        pltpu.sync_copy(prev_hbm.at[sl], pbuf)

        def body(i, _):
            for d in range(D // 16):
                s = pl.ds(d * 16, 16)
                obuf[i, s] = obuf[i, s] + bbuf[i, s] + pbuf[i, s]
            return 0

        lax.fori_loop(0, ROWS_CHUNK, body, 0, unroll=False)
        pltpu.sync_copy(obuf, out_hbm.at[sl])


def _edge_pass_l1_body(rows_hbm, cols_hbm, vals_hbm, tabu_hbm, tabv_hbm,
                       outu_hbm, outv_hbm, acc,
                       g0, s0, c0, v0, r0, i0, gs0, ss0,
                       g1, s1, c1, v1, r1, i1, gs1, ss1,
                       g2, s2, c2, v2, r2, i2, gs2, ss2, obuf):
    cid = lax.axis_index("c")
    sub = lax.axis_index("s")
    slots = ((g0, s0, c0, v0, r0, i0, gs0, ss0),
             (g1, s1, c1, v1, r1, i1, gs1, ss1),
             (g2, s2, c2, v2, r2, i2, gs2, ss2))

    @pl.when(cid == 0)
    def _():
        _edge_accumulate(tabu_hbm, cols_hbm, rows_hbm, vals_hbm, acc, slots,
                         obuf, sub)
        _write_out_plain(acc, obuf, outu_hbm, sub)

    @pl.when(cid == 1)
    def _():
        _edge_accumulate(tabv_hbm, rows_hbm, cols_hbm, vals_hbm, acc, slots,
                         obuf, sub)
        _write_out_plain(acc, obuf, outv_hbm, sub)


def _edge_pass_l2_body(rows_hbm, cols_hbm, vals_hbm, tabu_hbm, tabv_hbm,
                       baseu_hbm, basev_hbm, outu_hbm, outv_hbm, acc,
                       g0, s0, c0, v0, r0, i0, gs0, ss0,
                       g1, s1, c1, v1, r1, i1, gs1, ss1,
                       g2, s2, c2, v2, r2, i2, gs2, ss2, obuf, bbuf, pbuf):
    cid = lax.axis_index("c")
    sub = lax.axis_index("s")
    slots = ((g0, s0, c0, v0, r0, i0, gs0, ss0),
             (g1, s1, c1, v1, r1, i1, gs1, ss1),
             (g2, s2, c2, v2, r2, i2, gs2, ss2))

    @pl.when(cid == 0)
    def _():
        _edge_accumulate(tabu_hbm, cols_hbm, rows_hbm, vals_hbm, acc, slots,
                         obuf, sub)
        _write_out_fused(acc, obuf, bbuf, pbuf, baseu_hbm, tabv_hbm, outu_hbm,
                         sub)

    @pl.when(cid == 1)
    def _():
        _edge_accumulate(tabv_hbm, rows_hbm, cols_hbm, vals_hbm, acc, slots,
                         obuf, sub)
        _write_out_fused(acc, obuf, bbuf, pbuf, basev_hbm, tabu_hbm, outv_hbm,
                         sub)


def _slot_scratch():
    return [
        pltpu.VMEM((CH,), jnp.int32),
        pltpu.VMEM((CH,), jnp.int32),
        pltpu.VMEM((CH,), jnp.int32),
        pltpu.VMEM((CH,), jnp.float32),
        pltpu.VMEM((CH, D), jnp.float32),
        pltpu.SemaphoreType.DMA,
        pltpu.SemaphoreType.DMA,
        pltpu.SemaphoreType.DMA,
    ]


_SCRATCH_COMMON = (
    [pltpu.VMEM_SHARED((NP, D), jnp.float32)]
    + _slot_scratch() + _slot_scratch() + _slot_scratch()
    + [pltpu.VMEM((ROWS_CHUNK, D), jnp.float32)]
)

_edge_pass_l1 = pl.kernel(
    _edge_pass_l1_body,
    out_type=(jax.ShapeDtypeStruct((NP, D), jnp.float32),
              jax.ShapeDtypeStruct((NP, D), jnp.float32)),
    mesh=_mesh,
    scratch_types=_SCRATCH_COMMON,
    compiler_params=pltpu.CompilerParams(needs_layout_passes=False),
)

_edge_pass_l2 = pl.kernel(
    _edge_pass_l2_body,
    out_type=(jax.ShapeDtypeStruct((NP, D), jnp.float32),
              jax.ShapeDtypeStruct((NP, D), jnp.float32)),
    mesh=_mesh,
    scratch_types=_SCRATCH_COMMON + [
        pltpu.VMEM((ROWS_CHUNK, D), jnp.float32),
        pltpu.VMEM((ROWS_CHUNK, D), jnp.float32),
    ],
    compiler_params=pltpu.CompilerParams(needs_layout_passes=False),
)


def _aug_vals_body(rows_hbm, cols_hbm, adj_hbm, eu_hbm, ev_hbm, out_hbm,
                   r0, c0, a0, xu0, xi0, ob0, i0, gs0, os0,
                   r1, c1, a1, xu1, xi1, ob1, i1, gs1, os1, obig):
    wid = lax.axis_index("s") * NCORES + lax.axis_index("c")
    base = wid * E_PER_W
    slots = ((r0, c0, a0, xu0, xi0, ob0, i0, gs0, os0),
             (r1, c1, a1, xu1, xi1, ob1, i1, gs1, os1))

    def sync_idx(j, sl):
        ridx, cidx, abuf, xu, xi, obuf, isem, gsem, osem = sl
        off = base + j * CH
        pltpu.sync_copy(rows_hbm.at[pl.ds(off, CH)], ridx)
        pltpu.sync_copy(cols_hbm.at[pl.ds(off, CH)], cidx)
        pltpu.sync_copy(adj_hbm.at[pl.ds(off, CH)], abuf)

    def fire_idx(j, sl):
        ridx, cidx, abuf, xu, xi, obuf, isem, gsem, osem = sl
        off = base + j * CH
        pltpu.async_copy(rows_hbm.at[pl.ds(off, CH)], ridx, isem)
        pltpu.async_copy(cols_hbm.at[pl.ds(off, CH)], cidx, isem)
        pltpu.async_copy(adj_hbm.at[pl.ds(off, CH)], abuf, isem)

    def wait_idx(sl):
        ridx, cidx, abuf, xu, xi, obuf, isem, gsem, osem = sl
        pltpu.make_async_copy(rows_hbm.at[pl.ds(base, CH)], ridx, isem).wait()
        pltpu.make_async_copy(cols_hbm.at[pl.ds(base, CH)], cidx, isem).wait()
        pltpu.make_async_copy(adj_hbm.at[pl.ds(base, CH)], abuf, isem).wait()

    def fire_gather(sl):
        ridx, cidx, abuf, xu, xi, obuf, isem, gsem, osem = sl
        pltpu.async_copy(eu_hbm.at[ridx], xu, gsem)
        pltpu.async_copy(ev_hbm.at[cidx], xi, gsem)

    def wait_gather(sl):
        ridx, cidx, abuf, xu, xi, obuf, isem, gsem, osem = sl
        pltpu.make_async_copy(eu_hbm.at[ridx], xu, gsem).wait()
        pltpu.make_async_copy(ev_hbm.at[cidx], xi, gsem).wait()

    sync_idx(0, slots[0])
    sync_idx(1, slots[1])
    fire_gather(slots[0])
    lane = lax.iota(jnp.int32, 16)

    def pair(jj, _):
        for b in range(2):
            j = 2 * jj + b
            sl = slots[b]
            ot = slots[1 - b]
            ridx, cidx, abuf, xu, xi, obuf, isem, gsem, osem = sl

            @pl.when((j >= 1) & (j + 1 < NCH_W))
            def _():
                wait_idx(ot)

            @pl.when(j + 1 < NCH_W)
            def _():
                fire_gather(ot)

            wait_gather(sl)

            def group(g, _):
                def edge_fn(e16, dvec):
                    e = g * 16 + e16
                    acc = xu[e, pl.ds(0, 16)] * xi[e, pl.ds(0, 16)]
                    for d in range(1, D // 16):
                        s = pl.ds(d * 16, 16)
                        acc = acc + xu[e, s] * xi[e, s]
                    for k in (8, 4, 2, 1):
                        acc = acc + jnp.take(acc, lane ^ k)
                    return jnp.where(lane == e16, acc, dvec)

                dvec = lax.fori_loop(0, 16, edge_fn,
                                     jnp.zeros((16,), jnp.float32))
                obig[pl.ds((j % 5) * CH + g * 16, 16)] = (
                    abuf[pl.ds(g * 16, 16)] / (1.0 + jnp.exp(-dvec)))
                return 0

            lax.fori_loop(0, CH // 16, group, 0, unroll=False)

            @pl.when(j % 5 == 4)
            def _():
                pltpu.sync_copy(
                    obig, out_hbm.at[pl.ds(base + (j - 4) * CH, 5 * CH)])

            @pl.when(j + 2 < NCH_W)
            def _():
                fire_idx(j + 2, sl)
        return 0

    lax.fori_loop(0, NCH_W // 2, pair, 0, unroll=False)


def _aug_slot_scratch():
    return [
        pltpu.VMEM((CH,), jnp.int32),
        pltpu.VMEM((CH,), jnp.int32),
        pltpu.VMEM((CH,), jnp.float32),
        pltpu.VMEM((CH, D), jnp.float32),
        pltpu.VMEM((CH, D), jnp.float32),
        pltpu.VMEM((CH,), jnp.float32),
        pltpu.SemaphoreType.DMA,
        pltpu.SemaphoreType.DMA,
        pltpu.SemaphoreType.DMA,
    ]


_aug_vals = pl.kernel(
    _aug_vals_body,
    out_type=jax.ShapeDtypeStruct((E,), jnp.float32),
    mesh=_mesh,
    scratch_types=(_aug_slot_scratch() + _aug_slot_scratch()
                   + [pltpu.VMEM((5 * CH,), jnp.float32)]),
    compiler_params=pltpu.CompilerParams(needs_layout_passes=False),
)


def _gather6_body(eu_hbm, ev_hbm, zu_hbm, zv_hbm, uids_hbm, iids_hbm, pos_hbm,
                  neg_hbm, o_uemb, o_pos, o_neg, o_zub, o_zvb, o_evb,
                  ibuf, rbuf):
    wid = lax.axis_index("s") * NCORES + lax.axis_index("c")
    sl = pl.ds(wid * B_PER_W, B_PER_W)
    for idx_hbm, tab_hbm, out_hbm in (
        (uids_hbm, eu_hbm, o_uemb),
        (pos_hbm, ev_hbm, o_pos),
        (neg_hbm, ev_hbm, o_neg),
        (uids_hbm, zu_hbm, o_zub),
        (iids_hbm, zv_hbm, o_zvb),
        (iids_hbm, ev_hbm, o_evb),
    ):
        pltpu.sync_copy(idx_hbm.at[sl], ibuf)
        pltpu.sync_copy(tab_hbm.at[ibuf], rbuf)
        pltpu.sync_copy(rbuf, out_hbm.at[sl])


_gather6 = pl.kernel(
    _gather6_body,
    out_type=tuple(jax.ShapeDtypeStruct((B, D), jnp.float32)
                   for _ in range(6)),
    mesh=_mesh,
    scratch_types=[
        pltpu.VMEM((B_PER_W,), jnp.int32),
        pltpu.VMEM((B_PER_W, D), jnp.float32),
    ],
    compiler_params=pltpu.CompilerParams(needs_layout_passes=False),
)


def _losses_body(eu_ref, ev_ref, eu0_ref, ev0_ref, ebp_ref, uemb_ref, pos_ref,
                 neg_ref, zub_ref, zvb_ref, evb_ref, out_ref):
    u_emb = uemb_ref[...]
    pos_emb = pos_ref[...]
    neg_emb = neg_ref[...]
    zub = zub_ref[...]
    zvb = zvb_ref[...]
    evb = evb_ref[...]

    pos_scores = jnp.sum(u_emb * pos_emb, axis=1, keepdims=True)
    neg_scores = jnp.sum(u_emb * neg_emb, axis=1, keepdims=True)
    diff = pos_scores - neg_scores
    sig = 1.0 / (1.0 + jnp.exp(-diff))
    loss_bpr = -jnp.sum(jnp.log(sig)) / B

    def pcl_neg(zb, tab_ref):
        def blk(k, acc):
            t = tab_ref[pl.ds(k * 1000, 1000), :]
            s = lax.dot_general(zb, t, (((1,), (1,)), ((), ())),
                                preferred_element_type=jnp.float32)
            return acc + jnp.sum(jnp.exp(s / TEMP), axis=1, keepdims=True)

        acc = lax.fori_loop(0, N_U // 1000, blk,
                            jnp.zeros((B, 1), jnp.float32))
        return jnp.sum(jnp.log(acc + 1e-8)) / B

    neg_s = pcl_neg(zub, eu_ref) + pcl_neg(zvb, ev_ref)
    pos_s = (jnp.sum(jnp.clip(jnp.sum(zub * u_emb, axis=1) / TEMP, -5.0, 5.0))
             / B
             + jnp.sum(jnp.clip(jnp.sum(zvb * evb, axis=1) / TEMP, -5.0, 5.0))
             / B)
    loss_pcl = -pos_s + neg_s

    ps_min = jnp.min(pos_scores)
    ps_max = jnp.max(pos_scores)
    weight_b = (pos_scores - ps_min) / (ps_max - ps_min + 1e-9)
    relations = jnp.clip((weight_b * NB).astype(jnp.int32), 0, NB - 1)
    el = 1.0 / (1.0 + jnp.exp(-(u_emb * pos_emb)))
    s_all = lax.dot_general(el, ebp_ref[...], (((1,), (1,)), ((), ())),
                            preferred_element_type=jnp.float32)
    lane = lax.broadcasted_iota(jnp.int32, (B, NBP), 1)
    onehot = lane == relations
    srel = jnp.sum(jnp.where(onehot, s_all, 0.0), axis=1, keepdims=True)
    ssum = jnp.sum(s_all, axis=1, keepdims=True)
    neg_bcl = jnp.sum((ssum - srel) / NB) / B
    pos_bcl = jnp.sum(srel) / B
    loss_bcl = neg_bcl - pos_bcl

    def sq(tab_ref):
        def blk(k, acc):
            t = tab_ref[pl.ds(k * 200, 200), :]
            return acc + jnp.sum(t * t)

        return lax.fori_loop(0, N_U // 200, blk, jnp.float32(0.0))

    loss_reg = L3 * (sq(eu0_ref) + sq(ev0_ref) + jnp.sum(ebp_ref[...] ** 2))

    loss = loss_bpr + L1 * loss_pcl + L2 * loss_bcl + loss_reg
    out_ref[0] = loss
    out_ref[1] = loss_bpr
    out_ref[2] = L1 * loss_pcl
    out_ref[3] = L2 * loss_bcl


def _losses_call(eu, ev, eu0, ev0, ebp, uemb, posb, negb, zub, zvb, evb):
    return pl.pallas_call(
        _losses_body,
        out_shape=jax.ShapeDtypeStruct((4,), jnp.float32),
        in_specs=[pl.BlockSpec(memory_space=pltpu.VMEM)] * 11,
        out_specs=pl.BlockSpec(memory_space=pltpu.SMEM),
    )(eu, ev, eu0, ev0, ebp, uemb, posb, negb, zub, zvb, evb)


def kernel(E_u_0, E_v_0, E_b, adj_vals, edgE_vndex, uids, iids, pos, neg):
    rows = edgE_vndex[0]
    cols = edgE_vndex[1]
    pad = ((0, NP - N_U), (0, 0))
    eu0p = jnp.pad(E_u_0, pad)
    ev0p = jnp.pad(E_v_0, pad)

    nu1, nv1 = _edge_pass_l1(rows, cols, adj_vals, ev0p, eu0p)
    E_u, E_v = _edge_pass_l2(rows, cols, adj_vals, nv1, nu1, eu0p, ev0p)

    aug = _aug_vals(rows, cols, adj_vals, E_u, E_v)

    m_u1, m_v1 = _edge_pass_l1(rows, cols, aug, ev0p, eu0p)
    Z_u, Z_v = _edge_pass_l2(rows, cols, aug, m_v1, m_u1, eu0p, ev0p)

    u_emb, pos_emb, neg_emb, zub, zvb, evb = _gather6(
        E_u, E_v, Z_u, Z_v, uids, iids, pos, neg)

    ebp = jnp.zeros((NBP, D), jnp.float32).at[:NB].set(E_b)
    out = _losses_call(E_u, E_v, E_u_0, E_v_0, ebp, u_emb, pos_emb, neg_emb,
                       zub, zvb, evb)
    return (out[0], out[1], out[2], out[3])
